# baseline (speedup 1.0000x reference)
import functools

import jax
import jax.numpy as jnp
from jax import lax
from jax.experimental import pallas as pl
from jax.experimental.pallas import tpu as pltpu
from jax.experimental.pallas import tpu_sc as plsc

H = 256
H2 = 512
NEG_INF = float("-inf")

BN = 512
BE = 640

SC_CORES = 2
SC_SUBCORES = 16
SC_WORKERS = SC_CORES * SC_SUBCORES
GC = 40
SUMC = 40
MSUPER = 6400
MCH = 1280
NROW = 624
ZROW = 48


def _dg(x, w):
    return lax.dot_general(x, w, (((1,), (1,)), ((), ())),
                           preferred_element_type=jnp.float32)


def _dgt(xt, w):
    return lax.dot_general(xt, w, (((0,), (1,)), ((), ())),
                           preferred_element_type=jnp.float32)


def _lrelu(x):
    return jnp.where(x >= 0.0, x, 0.2 * x)


def _prep_body(nf, in1, wl1, bl1, wl2, bl2, wi, bi, wab, bab,
               x_o, xin1_o, ab_o):
    t = _dg(nf[...], wl1[...]) + bl1[...]
    x = _dg(t, wl2[...]) + bl2[...] + nf[...]
    x_o[...] = x
    xin1_o[...] = _dg(in1[...], wi[...]) + bi[...]
    ab_o[...] = _dg(x, wab[...]) + bab[...]


def _prep(nf, in1, rl1, rl2, wi, bi, wab, bab):
    n = nf.shape[0]
    grid = (pl.cdiv(n, BN),)
    row = lambda i: (i, 0)
    full = lambda i: (0, 0)
    return pl.pallas_call(
        _prep_body,
        grid=grid,
        in_specs=[
            pl.BlockSpec((BN, H), row), pl.BlockSpec((BN, H), row),
            pl.BlockSpec((H, H), full), pl.BlockSpec((1, H), full),
            pl.BlockSpec((H, H), full), pl.BlockSpec((1, H), full),
            pl.BlockSpec((H, H), full), pl.BlockSpec((1, H), full),
            pl.BlockSpec((H2, H), full), pl.BlockSpec((1, H2), full),
        ],
        out_specs=[
            pl.BlockSpec((BN, H), row), pl.BlockSpec((BN, H), row),
            pl.BlockSpec((BN, H2), row),
        ],
        out_shape=[
            jax.ShapeDtypeStruct((n, H), jnp.float32),
            jax.ShapeDtypeStruct((n, H), jnp.float32),
            jax.ShapeDtypeStruct((n, H2), jnp.float32),
        ],
    )(nf, in1, rl1[0], rl1[1].reshape(1, H), rl2[0], rl2[1].reshape(1, H),
      wi, bi.reshape(1, H), wab, bab.reshape(1, H2))


def _edge_body(pa, pb, wt, bt, ft_o):
    h = _lrelu(pa[...] + pb[...])
    t = _dg(h, wt[...]) + bt[...]
    k = jax.nn.sigmoid(t[:, H2:H2 + 1])
    ft_o[...] = jnp.transpose(t[:, :H2] * k)


def _edge(pre_a, pre_b, wt, bt):
    e = pre_a.shape[0]
    grid = (e // BE,)
    row = lambda i: (i, 0)
    col = lambda i: (0, i)
    full = lambda i: (0, 0)
    return pl.pallas_call(
        _edge_body,
        grid=grid,
        in_specs=[
            pl.BlockSpec((BE, H2), row), pl.BlockSpec((BE, H2), row),
            pl.BlockSpec((640, H2), full), pl.BlockSpec((1, 640), full),
        ],
        out_specs=pl.BlockSpec((H2, BE), col),
        out_shape=jax.ShapeDtypeStruct((H2, e), jnp.float32),
    )(pre_a, pre_b, wt, bt)


def _post_body(x_dst, f1st, mxt, x_in1,
               wr1, wr2, wr3, br, wg, bg, wpc1, wpc2, bpc, out_o):
    nfno2t = jnp.where(mxt[...] == NEG_INF, 0.0, mxt[...])
    new_x = (_dg(x_dst[...], wr1[...]) + _dgt(f1st[...], wr2[...]) +
             _dgt(nfno2t, wr3[...]) + br[...])
    new_x = _dg(new_x, wg[...]) + bg[...]
    out_o[...] = (x_dst[...] + _dg(new_x, wpc1[...]) +
                  _dg(x_in1[...], wpc2[...]) + bpc[...])


def _post(x_dst, f1st, mxt, x_in1, red, gw, gb, pcw, pcb):
    n = x_dst.shape[0]
    grid = (pl.cdiv(n, BN),)
    row = lambda i: (i, 0)
    col = lambda i: (0, i)
    full = lambda i: (0, 0)
    wred, bred = red
    return pl.pallas_call(
        _post_body,
        grid=grid,
        in_specs=[
            pl.BlockSpec((BN, H), row), pl.BlockSpec((H, BN), col),
            pl.BlockSpec((H, BN), col), pl.BlockSpec((BN, H), row),
            pl.BlockSpec((H, H), full), pl.BlockSpec((H, H), full),
            pl.BlockSpec((H, H), full), pl.BlockSpec((1, H), full),
            pl.BlockSpec((H, H), full), pl.BlockSpec((1, H), full),
            pl.BlockSpec((H, H), full), pl.BlockSpec((H, H), full),
            pl.BlockSpec((1, H), full),
        ],
        out_specs=pl.BlockSpec((BN, H), row),
        out_shape=jax.ShapeDtypeStruct((n, H), jnp.float32),
    )(x_dst, f1st, mxt, x_in1,
      wred[:, :H], wred[:, H:2 * H], wred[:, 2 * H:], bred.reshape(1, H),
      gw, gb.reshape(1, H), pcw[:, :H], pcw[:, H:], pcb.reshape(1, H))


def _mid_body(xn, xc, nl1w, nl1b, nl2w, nl2b, cl1w, cl1b, cl2w, cl2b,
              w1s, w1d, b1, xn_o, a_o, xc_o, b_o):
    t = _dg(xn[...], nl1w[...]) + nl1b[...]
    xnn = _dg(t, nl2w[...]) + nl2b[...] + xn[...]
    xn_o[...] = xnn
    a_o[...] = _dg(xnn, w1s[...])
    t2 = _dg(xc[...], cl1w[...]) + cl1b[...]
    xcc = _dg(t2, cl2w[...]) + cl2b[...] + xc[...]
    xc_o[...] = xcc
    b_o[...] = _dg(xcc, w1d[...]) + b1[...]


def _mid(x_gn_mid, x_gc, res_gn2, res_gc2, w1s, w1d, b1):
    n = x_gn_mid.shape[0]
    grid = (pl.cdiv(n, BN),)
    row = lambda i: (i, 0)
    full = lambda i: (0, 0)
    return pl.pallas_call(
        _mid_body,
        grid=grid,
        in_specs=[
            pl.BlockSpec((BN, H), row), pl.BlockSpec((BN, H), row),
            pl.BlockSpec((H, H), full), pl.BlockSpec((1, H), full),
            pl.BlockSpec((H, H), full), pl.BlockSpec((1, H), full),
            pl.BlockSpec((H, H), full), pl.BlockSpec((1, H), full),
            pl.BlockSpec((H, H), full), pl.BlockSpec((1, H), full),
            pl.BlockSpec((H2, H), full), pl.BlockSpec((H2, H), full),
            pl.BlockSpec((1, H2), full),
        ],
        out_specs=[
            pl.BlockSpec((BN, H), row), pl.BlockSpec((BN, H2), row),
            pl.BlockSpec((BN, H), row), pl.BlockSpec((BN, H2), row),
        ],
        out_shape=[
            jax.ShapeDtypeStruct((n, H), jnp.float32),
            jax.ShapeDtypeStruct((n, H2), jnp.float32),
            jax.ShapeDtypeStruct((n, H), jnp.float32),
            jax.ShapeDtypeStruct((n, H2), jnp.float32),
        ],
    )(x_gn_mid, x_gc,
      res_gn2['l1'][0], res_gn2['l1'][1].reshape(1, H),
      res_gn2['l2'][0], res_gn2['l2'][1].reshape(1, H),
      res_gc2['l1'][0], res_gc2['l1'][1].reshape(1, H),
      res_gc2['l2'][0], res_gc2['l2'][1].reshape(1, H),
      w1s, w1d, b1.reshape(1, H2))


def _sc_gather(a_tab, b_tab, src, dst):
    e = src.shape[0]
    ep = e // SC_WORKERS
    chunks = ep // GC
    mesh = plsc.VectorSubcoreMesh(core_axis_name="c", subcore_axis_name="s")

    @functools.partial(
        pl.kernel, mesh=mesh,
        out_type=[
            jax.ShapeDtypeStruct((e, H2), jnp.float32),
            jax.ShapeDtypeStruct((e, H2), jnp.float32),
        ],
        compiler_params=pltpu.CompilerParams(needs_layout_passes=False),
        scratch_types=[
            pltpu.VMEM((GC,), jnp.int32),
            pltpu.VMEM((GC,), jnp.int32),
            pltpu.VMEM((GC, H2), jnp.float32),
            pltpu.VMEM((GC, H2), jnp.float32),
            pltpu.SemaphoreType.DMA,
        ],
    )
    def k(a_h, b_h, src_h, dst_h, pa_h, pb_h, ia, ib, ra, rb, sem):
        c = lax.axis_index("c")
        s = lax.axis_index("s")
        wid = s * SC_CORES + c

        def chunk(i, carry):
            base = pl.multiple_of(wid * ep + i * GC, 8)
            pltpu.sync_copy(src_h.at[pl.ds(base, GC)], ia)
            pltpu.sync_copy(dst_h.at[pl.ds(base, GC)], ib)
            cp1 = pltpu.async_copy(a_h.at[ia], ra, sem)
            cp1.wait()
            cp2 = pltpu.async_copy(b_h.at[ib], rb, sem)
            cp2.wait()
            pltpu.sync_copy(ra, pa_h.at[pl.ds(base, GC)])
            pltpu.sync_copy(rb, pb_h.at[pl.ds(base, GC)])
            return carry

        lax.fori_loop(0, chunks, chunk, 0)

    return k(a_tab, b_tab, src, dst)


def _sc_scatter_sum(ftt, dst, n):
    e = dst.shape[0]
    chunks = e // MCH
    groups = MCH // 16
    mesh = plsc.VectorSubcoreMesh(core_axis_name="c", subcore_axis_name="s")

    @functools.partial(
        pl.kernel, mesh=mesh,
        out_type=jax.ShapeDtypeStruct((H * n,), jnp.float32),
        compiler_params=pltpu.CompilerParams(needs_layout_passes=False),
        scratch_types=[
            pltpu.VMEM((8 * n,), jnp.float32),
            pltpu.VMEM((n,), jnp.int32),
            pltpu.VMEM((MCH,), jnp.int32),
            pltpu.VMEM((8, MCH), jnp.float32),
        ],
    )
    def k(ftt_h, dst_h, out_h, acc, tag, midx, mval):
        c = lax.axis_index("c")
        s = lax.axis_index("s")
        tid = c * SC_SUBCORES + s
        iota = lax.iota(jnp.int32, 16)

        def minit(i, carry):
            acc[pl.ds(i * 16, 16)] = jnp.zeros((16,), jnp.float32)
            return carry
        lax.fori_loop(0, (8 * n) // 16, minit, 0)

        def ch_loop(ch, carry):
            base = ch * MCH
            pltpu.sync_copy(dst_h.at[pl.ds(base, MCH)], midx)
            pltpu.sync_copy(
                ftt_h.at[pl.ds(H + tid * 8, 8), pl.ds(base, MCH)], mval)

            def grp(q, carry3):
                for u in range(4):
                    g = q * 4 + u
                    dstv = midx[pl.ds(g * 16, 16)]
                    for f in range(8):
                        plsc.addupdate_scatter(
                            acc, [dstv + f * n], mval[f, pl.ds(g * 16, 16)])
                return carry3
            lax.fori_loop(0, groups // 4, grp, 0)
            return carry
        lax.fori_loop(0, chunks, ch_loop, 0)

        for f in range(8):
            pltpu.sync_copy(acc.at[pl.ds(f * n, n)],
                            out_h.at[pl.ds((tid * 8 + f) * n, n)])

    return k(ftt, dst)


def _sc_scatter_max(ftt, dst, n):
    e = dst.shape[0]
    chunks = e // MCH
    groups = MCH // 16
    mesh = plsc.VectorSubcoreMesh(core_axis_name="c", subcore_axis_name="s")

    @functools.partial(
        pl.kernel, mesh=mesh,
        out_type=jax.ShapeDtypeStruct((H * n,), jnp.float32),
        compiler_params=pltpu.CompilerParams(needs_layout_passes=False),
        scratch_types=[
            pltpu.VMEM((8 * n,), jnp.float32),
            pltpu.VMEM((n,), jnp.int32),
            pltpu.VMEM((MCH,), jnp.int32),
            pltpu.VMEM((8, MCH), jnp.float32),
        ],
    )
    def k(ftt_h, dst_h, out_h, acc, tag, midx, mval):
        c = lax.axis_index("c")
        s = lax.axis_index("s")
        tid = c * SC_SUBCORES + s
        iota = lax.iota(jnp.int32, 16)

        def minit(i, carry):
            acc[pl.ds(i * 16, 16)] = jnp.full((16,), NEG_INF, jnp.float32)
            return carry
        lax.fori_loop(0, (8 * n) // 16, minit, 0)

        def ch_loop(ch, carry):
            base = ch * MCH
            pltpu.sync_copy(dst_h.at[pl.ds(base, MCH)], midx)
            pltpu.sync_copy(
                ftt_h.at[pl.ds(tid * 8, 8), pl.ds(base, MCH)], mval)

            def grp(q, carry3):
                dstvs = [midx[pl.ds((q * 4 + u) * 16, 16)] for u in range(4)]
                ok = jnp.full((16,), True)
                for u in range(4):
                    plsc.store_scatter(tag, [dstvs[u]], iota + u * 16)
                for u in range(4):
                    got = plsc.load_gather(tag, [dstvs[u]])
                    ok = ok & (got == iota + u * 16)
                dup_free = jnp.all(ok)

                @pl.when(dup_free)
                def _():
                    for u in range(4):
                        g = q * 4 + u
                        dstv = dstvs[u]
                        for f in range(8):
                            val = mval[f, pl.ds(g * 16, 16)]
                            aidx = dstv + f * n
                            cur = plsc.load_gather(acc, [aidx])
                            plsc.store_scatter(acc, [aidx],
                                               jnp.maximum(cur, val))

                @pl.when(jnp.logical_not(dup_free))
                def _():
                    for u in range(4):
                        g = q * 4 + u
                        dstv = dstvs[u]
                        for f in range(8):
                            val = mval[f, pl.ds(g * 16, 16)]
                            aidx = dstv + f * n
                            cur = plsc.load_gather(acc, [aidx])
                            m0 = val > cur

                            def wcond(m):
                                return jnp.any(m)

                            def wbody(m):
                                plsc.store_scatter(acc, [aidx], val, mask=m)
                                cur2 = plsc.load_gather(acc, [aidx])
                                return m & (cur2 < val)

                            lax.while_loop(wcond, wbody, m0)
                return carry3
            lax.fori_loop(0, groups // 4, grp, 0)
            return carry
        lax.fori_loop(0, chunks, ch_loop, 0)

        for f in range(8):
            pltpu.sync_copy(acc.at[pl.ds(f * n, n)],
                            out_h.at[pl.ds((tid * 8 + f) * n, n)])

    return k(ftt, dst)


def _direction(a_tab, b_tab, x_dst, x_in1, edge, msg, red, gw, gb, pcw, pcb,
               n_dst):
    w2, b2 = msg['l2']
    wt = jnp.concatenate(
        [w2[1 + H:], w2[1:1 + H], w2[0:1],
         jnp.zeros((127, H2), jnp.float32)], axis=0)
    bt = jnp.concatenate(
        [b2[1 + H:], b2[1:1 + H], b2[0:1],
         jnp.zeros((127,), jnp.float32)]).reshape(1, 640)
    src, dst = edge[0], edge[1]
    pre_a, pre_b = _sc_gather(a_tab, b_tab, src, dst)
    ftt = _edge(pre_a, pre_b, wt, bt)
    f1st = _sc_scatter_sum(ftt, dst, n_dst).reshape(H, n_dst)
    mxt = _sc_scatter_max(ftt, dst, n_dst).reshape(H, n_dst)
    return _post(x_dst, f1st, mxt, x_in1, red, gw, gb, pcw, pcb)


def kernel(nf_gc, nf_gn, nf_gc_in1, nf_gn_in1, edge_c2n, edge_n2c, params):
    p = params
    w1c, b1c = p['msg_c2n']['l1']
    w1n, b1n = p['msg_n2c']['l1']

    x_gc, x_gc_in1, a_c2n = _prep(
        nf_gc, nf_gc_in1, p['res_gc_1']['l1'], p['res_gc_1']['l2'],
        p['gc_in1'][0], p['gc_in1'][1], w1c[:, :H],
        jnp.zeros((H2,), jnp.float32))
    x_gn, x_gn_in1, b_c2n = _prep(
        nf_gn, nf_gn_in1, p['res_gn_1']['l1'], p['res_gn_1']['l2'],
        p['gn_in1'][0], p['gn_in1'][1], w1c[:, H:], b1c)

    x_gn_mid = _direction(a_c2n, b_c2n, x_gn, x_gn_in1, edge_c2n,
                          p['msg_c2n'], p['red_c2n'], p['Gcn'][0],
                          p['Gcn'][1], p['postCatGcn'][0], p['postCatGcn'][1],
                          nf_gn.shape[0])

    x_gn2, a_n2c, x_gc2, b_n2c = _mid(
        x_gn_mid, x_gc, p['res_gn_2'], p['res_gc_2'],
        w1n[:, :H], w1n[:, H:], b1n)

    x_gc_out = _direction(a_n2c, b_n2c, x_gc2, x_gc_in1, edge_n2c,
                          p['msg_n2c'], p['red_n2c'], p['Gnc'][0],
                          p['Gnc'][1], p['postCatGnc'][0], p['postCatGnc'][1],
                          nf_gc.shape[0])

    return (x_gc_out, x_gn2)


# speedup vs baseline: 1.1175x; 1.1175x over previous
import functools

import jax
import jax.numpy as jnp
from jax import lax
from jax.experimental import pallas as pl
from jax.experimental.pallas import tpu as pltpu
from jax.experimental.pallas import tpu_sc as plsc

H = 256
H2 = 512
NEG_INF = float("-inf")

BN = 512
BE = 640

SC_CORES = 2
SC_SUBCORES = 16
SC_WORKERS = SC_CORES * SC_SUBCORES
GC = 40
SUMC = 40
MSUPER = 6400
MCH = 1280
NROW = 624
ZROW = 48


def _dg(x, w):
    return lax.dot_general(x, w, (((1,), (1,)), ((), ())),
                           preferred_element_type=jnp.float32)


def _dgt(xt, w):
    return lax.dot_general(xt, w, (((0,), (1,)), ((), ())),
                           preferred_element_type=jnp.float32)


def _lrelu(x):
    return jnp.where(x >= 0.0, x, 0.2 * x)


def _prep_body(nf, in1, wl1, bl1, wl2, bl2, wi, bi, wab, bab,
               x_o, xin1_o, ab_o):
    t = _dg(nf[...], wl1[...]) + bl1[...]
    x = _dg(t, wl2[...]) + bl2[...] + nf[...]
    x_o[...] = x
    xin1_o[...] = _dg(in1[...], wi[...]) + bi[...]
    ab_o[...] = _dg(x, wab[...]) + bab[...]


def _prep(nf, in1, rl1, rl2, wi, bi, wab, bab):
    n = nf.shape[0]
    grid = (pl.cdiv(n, BN),)
    row = lambda i: (i, 0)
    full = lambda i: (0, 0)
    return pl.pallas_call(
        _prep_body,
        grid=grid,
        in_specs=[
            pl.BlockSpec((BN, H), row), pl.BlockSpec((BN, H), row),
            pl.BlockSpec((H, H), full), pl.BlockSpec((1, H), full),
            pl.BlockSpec((H, H), full), pl.BlockSpec((1, H), full),
            pl.BlockSpec((H, H), full), pl.BlockSpec((1, H), full),
            pl.BlockSpec((H2, H), full), pl.BlockSpec((1, H2), full),
        ],
        out_specs=[
            pl.BlockSpec((BN, H), row), pl.BlockSpec((BN, H), row),
            pl.BlockSpec((BN, H2), row),
        ],
        out_shape=[
            jax.ShapeDtypeStruct((n, H), jnp.float32),
            jax.ShapeDtypeStruct((n, H), jnp.float32),
            jax.ShapeDtypeStruct((n, H2), jnp.float32),
        ],
    )(nf, in1, rl1[0], rl1[1].reshape(1, H), rl2[0], rl2[1].reshape(1, H),
      wi, bi.reshape(1, H), wab, bab.reshape(1, H2))


def _edge_body(pa, pb, wt, bt, ft_o):
    h = _lrelu(pa[...] + pb[...])
    t = _dg(h, wt[...]) + bt[...]
    k = jax.nn.sigmoid(t[:, H2:H2 + 1])
    ft_o[...] = jnp.transpose(t[:, :H2] * k)


def _edge(pre_a, pre_b, wt, bt):
    e = pre_a.shape[0]
    grid = (e // BE,)
    row = lambda i: (i, 0)
    col = lambda i: (0, i)
    full = lambda i: (0, 0)
    return pl.pallas_call(
        _edge_body,
        grid=grid,
        in_specs=[
            pl.BlockSpec((BE, H2), row), pl.BlockSpec((BE, H2), row),
            pl.BlockSpec((640, H2), full), pl.BlockSpec((1, 640), full),
        ],
        out_specs=pl.BlockSpec((H2, BE), col),
        out_shape=jax.ShapeDtypeStruct((H2, e), jnp.float32),
    )(pre_a, pre_b, wt, bt)


def _post_body(x_dst, f1st, mxt, x_in1,
               wr1, wr2, wr3, br, wg, bg, wpc1, wpc2, bpc, out_o):
    nfno2t = jnp.where(mxt[...] == NEG_INF, 0.0, mxt[...])
    new_x = (_dg(x_dst[...], wr1[...]) + _dgt(f1st[...], wr2[...]) +
             _dgt(nfno2t, wr3[...]) + br[...])
    new_x = _dg(new_x, wg[...]) + bg[...]
    out_o[...] = (x_dst[...] + _dg(new_x, wpc1[...]) +
                  _dg(x_in1[...], wpc2[...]) + bpc[...])


def _post(x_dst, f1st, mxt, x_in1, red, gw, gb, pcw, pcb):
    n = x_dst.shape[0]
    grid = (pl.cdiv(n, BN),)
    row = lambda i: (i, 0)
    col = lambda i: (0, i)
    full = lambda i: (0, 0)
    wred, bred = red
    return pl.pallas_call(
        _post_body,
        grid=grid,
        in_specs=[
            pl.BlockSpec((BN, H), row), pl.BlockSpec((H, BN), col),
            pl.BlockSpec((H, BN), col), pl.BlockSpec((BN, H), row),
            pl.BlockSpec((H, H), full), pl.BlockSpec((H, H), full),
            pl.BlockSpec((H, H), full), pl.BlockSpec((1, H), full),
            pl.BlockSpec((H, H), full), pl.BlockSpec((1, H), full),
            pl.BlockSpec((H, H), full), pl.BlockSpec((H, H), full),
            pl.BlockSpec((1, H), full),
        ],
        out_specs=pl.BlockSpec((BN, H), row),
        out_shape=jax.ShapeDtypeStruct((n, H), jnp.float32),
    )(x_dst, f1st, mxt, x_in1,
      wred[:, :H], wred[:, H:2 * H], wred[:, 2 * H:], bred.reshape(1, H),
      gw, gb.reshape(1, H), pcw[:, :H], pcw[:, H:], pcb.reshape(1, H))


def _mid_body(xn, xc, nl1w, nl1b, nl2w, nl2b, cl1w, cl1b, cl2w, cl2b,
              w1s, w1d, b1, xn_o, a_o, xc_o, b_o):
    t = _dg(xn[...], nl1w[...]) + nl1b[...]
    xnn = _dg(t, nl2w[...]) + nl2b[...] + xn[...]
    xn_o[...] = xnn
    a_o[...] = _dg(xnn, w1s[...])
    t2 = _dg(xc[...], cl1w[...]) + cl1b[...]
    xcc = _dg(t2, cl2w[...]) + cl2b[...] + xc[...]
    xc_o[...] = xcc
    b_o[...] = _dg(xcc, w1d[...]) + b1[...]


def _mid(x_gn_mid, x_gc, res_gn2, res_gc2, w1s, w1d, b1):
    n = x_gn_mid.shape[0]
    grid = (pl.cdiv(n, BN),)
    row = lambda i: (i, 0)
    full = lambda i: (0, 0)
    return pl.pallas_call(
        _mid_body,
        grid=grid,
        in_specs=[
            pl.BlockSpec((BN, H), row), pl.BlockSpec((BN, H), row),
            pl.BlockSpec((H, H), full), pl.BlockSpec((1, H), full),
            pl.BlockSpec((H, H), full), pl.BlockSpec((1, H), full),
            pl.BlockSpec((H, H), full), pl.BlockSpec((1, H), full),
            pl.BlockSpec((H, H), full), pl.BlockSpec((1, H), full),
            pl.BlockSpec((H2, H), full), pl.BlockSpec((H2, H), full),
            pl.BlockSpec((1, H2), full),
        ],
        out_specs=[
            pl.BlockSpec((BN, H), row), pl.BlockSpec((BN, H2), row),
            pl.BlockSpec((BN, H), row), pl.BlockSpec((BN, H2), row),
        ],
        out_shape=[
            jax.ShapeDtypeStruct((n, H), jnp.float32),
            jax.ShapeDtypeStruct((n, H2), jnp.float32),
            jax.ShapeDtypeStruct((n, H), jnp.float32),
            jax.ShapeDtypeStruct((n, H2), jnp.float32),
        ],
    )(x_gn_mid, x_gc,
      res_gn2['l1'][0], res_gn2['l1'][1].reshape(1, H),
      res_gn2['l2'][0], res_gn2['l2'][1].reshape(1, H),
      res_gc2['l1'][0], res_gc2['l1'][1].reshape(1, H),
      res_gc2['l2'][0], res_gc2['l2'][1].reshape(1, H),
      w1s, w1d, b1.reshape(1, H2))


def _sc_gather(a_tab, b_tab, src, dst):
    e = src.shape[0]
    ep = e // SC_WORKERS
    chunks = ep // GC
    mesh = plsc.VectorSubcoreMesh(core_axis_name="c", subcore_axis_name="s")

    @functools.partial(
        pl.kernel, mesh=mesh,
        out_type=[
            jax.ShapeDtypeStruct((e, H2), jnp.float32),
            jax.ShapeDtypeStruct((e, H2), jnp.float32),
        ],
        compiler_params=pltpu.CompilerParams(needs_layout_passes=False),
        scratch_types=[
            pltpu.VMEM((GC,), jnp.int32),
            pltpu.VMEM((GC,), jnp.int32),
            pltpu.VMEM((GC, H2), jnp.float32),
            pltpu.VMEM((GC, H2), jnp.float32),
            pltpu.SemaphoreType.DMA,
        ],
    )
    def k(a_h, b_h, src_h, dst_h, pa_h, pb_h, ia, ib, ra, rb, sem):
        c = lax.axis_index("c")
        s = lax.axis_index("s")
        wid = s * SC_CORES + c

        def chunk(i, carry):
            base = pl.multiple_of(wid * ep + i * GC, 8)
            pltpu.sync_copy(src_h.at[pl.ds(base, GC)], ia)
            pltpu.sync_copy(dst_h.at[pl.ds(base, GC)], ib)
            cp1 = pltpu.async_copy(a_h.at[ia], ra, sem)
            cp1.wait()
            cp2 = pltpu.async_copy(b_h.at[ib], rb, sem)
            cp2.wait()
            pltpu.sync_copy(ra, pa_h.at[pl.ds(base, GC)])
            pltpu.sync_copy(rb, pb_h.at[pl.ds(base, GC)])
            return carry

        lax.fori_loop(0, chunks, chunk, 0)

    return k(a_tab, b_tab, src, dst)


def _sc_scatter_sum(ftt, dst, n):
    e = dst.shape[0]
    chunks = e // MCH
    groups = MCH // 16
    mesh = plsc.VectorSubcoreMesh(core_axis_name="c", subcore_axis_name="s")

    @functools.partial(
        pl.kernel, mesh=mesh,
        out_type=jax.ShapeDtypeStruct((H * n,), jnp.float32),
        compiler_params=pltpu.CompilerParams(needs_layout_passes=False),
        scratch_types=[
            pltpu.VMEM((8 * n,), jnp.float32),
            pltpu.VMEM((n,), jnp.int32),
            pltpu.VMEM((MCH,), jnp.int32),
            pltpu.VMEM((8, MCH), jnp.float32),
        ],
    )
    def k(ftt_h, dst_h, out_h, acc, tag, midx, mval):
        c = lax.axis_index("c")
        s = lax.axis_index("s")
        tid = c * SC_SUBCORES + s
        iota = lax.iota(jnp.int32, 16)

        def minit(i, carry):
            acc[pl.ds(i * 16, 16)] = jnp.zeros((16,), jnp.float32)
            return carry
        lax.fori_loop(0, (8 * n) // 16, minit, 0)

        def ch_loop(ch, carry):
            base = ch * MCH
            pltpu.sync_copy(dst_h.at[pl.ds(base, MCH)], midx)
            pltpu.sync_copy(
                ftt_h.at[pl.ds(H + tid * 8, 8), pl.ds(base, MCH)], mval)

            def grp(q, carry3):
                for u in range(4):
                    g = q * 4 + u
                    dstv = midx[pl.ds(g * 16, 16)]
                    for f in range(8):
                        plsc.addupdate_scatter(
                            acc, [dstv + f * n], mval[f, pl.ds(g * 16, 16)])
                return carry3
            lax.fori_loop(0, groups // 4, grp, 0)
            return carry
        lax.fori_loop(0, chunks, ch_loop, 0)

        for f in range(8):
            pltpu.sync_copy(acc.at[pl.ds(f * n, n)],
                            out_h.at[pl.ds((tid * 8 + f) * n, n)])

    return k(ftt, dst)


def _sc_scatter_max(ftt, dst, n):
    e = dst.shape[0]
    chunks = e // MCH
    groups = MCH // 16
    mesh = plsc.VectorSubcoreMesh(core_axis_name="c", subcore_axis_name="s")

    @functools.partial(
        pl.kernel, mesh=mesh,
        out_type=jax.ShapeDtypeStruct((H * n,), jnp.float32),
        compiler_params=pltpu.CompilerParams(needs_layout_passes=False),
        scratch_types=[
            pltpu.VMEM((8 * n,), jnp.float32),
            pltpu.VMEM((n,), jnp.int32),
            pltpu.VMEM((MCH,), jnp.int32),
            pltpu.VMEM((8, MCH), jnp.float32),
        ],
    )
    def k(ftt_h, dst_h, out_h, acc, tag, midx, mval):
        c = lax.axis_index("c")
        s = lax.axis_index("s")
        tid = c * SC_SUBCORES + s
        iota = lax.iota(jnp.int32, 16)

        def minit(i, carry):
            acc[pl.ds(i * 16, 16)] = jnp.full((16,), NEG_INF, jnp.float32)
            return carry
        lax.fori_loop(0, (8 * n) // 16, minit, 0)

        def ch_loop(ch, carry):
            base = ch * MCH
            pltpu.sync_copy(dst_h.at[pl.ds(base, MCH)], midx)
            pltpu.sync_copy(
                ftt_h.at[pl.ds(tid * 8, 8), pl.ds(base, MCH)], mval)

            def grp(g, carry3):
                dstv = midx[pl.ds(g * 16, 16)]
                plsc.store_scatter(tag, [dstv], iota)
                dup_free = jnp.all(plsc.load_gather(tag, [dstv]) == iota)

                @pl.when(dup_free)
                def _():
                    for f in range(8):
                        val = mval[f, pl.ds(g * 16, 16)]
                        aidx = dstv + f * n
                        cur = plsc.load_gather(acc, [aidx])
                        plsc.store_scatter(acc, [aidx],
                                           jnp.maximum(cur, val))

                @pl.when(jnp.logical_not(dup_free))
                def _():
                    for f in range(8):
                        val = mval[f, pl.ds(g * 16, 16)]
                        aidx = dstv + f * n
                        cur = plsc.load_gather(acc, [aidx])
                        m0 = val > cur

                        def wcond(m):
                            return jnp.any(m)

                        def wbody(m):
                            plsc.store_scatter(acc, [aidx], val, mask=m)
                            cur2 = plsc.load_gather(acc, [aidx])
                            return m & (cur2 < val)

                        lax.while_loop(wcond, wbody, m0)
                return carry3
            lax.fori_loop(0, groups, grp, 0)
            return carry
        lax.fori_loop(0, chunks, ch_loop, 0)

        for f in range(8):
            pltpu.sync_copy(acc.at[pl.ds(f * n, n)],
                            out_h.at[pl.ds((tid * 8 + f) * n, n)])

    return k(ftt, dst)


def _direction(a_tab, b_tab, x_dst, x_in1, edge, msg, red, gw, gb, pcw, pcb,
               n_dst):
    w2, b2 = msg['l2']
    wt = jnp.concatenate(
        [w2[1 + H:], w2[1:1 + H], w2[0:1],
         jnp.zeros((127, H2), jnp.float32)], axis=0)
    bt = jnp.concatenate(
        [b2[1 + H:], b2[1:1 + H], b2[0:1],
         jnp.zeros((127,), jnp.float32)]).reshape(1, 640)
    src, dst = edge[0], edge[1]
    pre_a, pre_b = _sc_gather(a_tab, b_tab, src, dst)
    ftt = _edge(pre_a, pre_b, wt, bt)
    f1st = _sc_scatter_sum(ftt, dst, n_dst).reshape(H, n_dst)
    mxt = _sc_scatter_max(ftt, dst, n_dst).reshape(H, n_dst)
    return _post(x_dst, f1st, mxt, x_in1, red, gw, gb, pcw, pcb)


def kernel(nf_gc, nf_gn, nf_gc_in1, nf_gn_in1, edge_c2n, edge_n2c, params):
    p = params
    w1c, b1c = p['msg_c2n']['l1']
    w1n, b1n = p['msg_n2c']['l1']

    x_gc, x_gc_in1, a_c2n = _prep(
        nf_gc, nf_gc_in1, p['res_gc_1']['l1'], p['res_gc_1']['l2'],
        p['gc_in1'][0], p['gc_in1'][1], w1c[:, :H],
        jnp.zeros((H2,), jnp.float32))
    x_gn, x_gn_in1, b_c2n = _prep(
        nf_gn, nf_gn_in1, p['res_gn_1']['l1'], p['res_gn_1']['l2'],
        p['gn_in1'][0], p['gn_in1'][1], w1c[:, H:], b1c)

    x_gn_mid = _direction(a_c2n, b_c2n, x_gn, x_gn_in1, edge_c2n,
                          p['msg_c2n'], p['red_c2n'], p['Gcn'][0],
                          p['Gcn'][1], p['postCatGcn'][0], p['postCatGcn'][1],
                          nf_gn.shape[0])

    x_gn2, a_n2c, x_gc2, b_n2c = _mid(
        x_gn_mid, x_gc, p['res_gn_2'], p['res_gc_2'],
        w1n[:, :H], w1n[:, H:], b1n)

    x_gc_out = _direction(a_n2c, b_n2c, x_gc2, x_gc_in1, edge_n2c,
                          p['msg_n2c'], p['red_n2c'], p['Gnc'][0],
                          p['Gnc'][1], p['postCatGnc'][0], p['postCatGnc'][1],
                          nf_gc.shape[0])

    return (x_gc_out, x_gn2)


# speedup vs baseline: 1.1738x; 1.0504x over previous
import functools

import jax
import jax.numpy as jnp
from jax import lax
from jax.experimental import pallas as pl
from jax.experimental.pallas import tpu as pltpu
from jax.experimental.pallas import tpu_sc as plsc

H = 256
H2 = 512
NEG_INF = float("-inf")

BN = 512
BE = 640

SC_CORES = 2
SC_SUBCORES = 16
SC_WORKERS = SC_CORES * SC_SUBCORES
GC = 40
SUMC = 40
MSUPER = 6400
MCH = 1280
NROW = 624
ZROW = 48


def _dg(x, w):
    return lax.dot_general(x, w, (((1,), (1,)), ((), ())),
                           preferred_element_type=jnp.float32)


def _dgt(xt, w):
    return lax.dot_general(xt, w, (((0,), (1,)), ((), ())),
                           preferred_element_type=jnp.float32)


def _lrelu(x):
    return jnp.where(x >= 0.0, x, 0.2 * x)


def _prep_body(nf, in1, wl1, bl1, wl2, bl2, wi, bi, wab, bab,
               x_o, xin1_o, ab_o):
    t = _dg(nf[...], wl1[...]) + bl1[...]
    x = _dg(t, wl2[...]) + bl2[...] + nf[...]
    x_o[...] = x
    xin1_o[...] = _dg(in1[...], wi[...]) + bi[...]
    ab_o[...] = _dg(x, wab[...]) + bab[...]


def _prep(nf, in1, rl1, rl2, wi, bi, wab, bab):
    n = nf.shape[0]
    grid = (pl.cdiv(n, BN),)
    row = lambda i: (i, 0)
    full = lambda i: (0, 0)
    return pl.pallas_call(
        _prep_body,
        grid=grid,
        in_specs=[
            pl.BlockSpec((BN, H), row), pl.BlockSpec((BN, H), row),
            pl.BlockSpec((H, H), full), pl.BlockSpec((1, H), full),
            pl.BlockSpec((H, H), full), pl.BlockSpec((1, H), full),
            pl.BlockSpec((H, H), full), pl.BlockSpec((1, H), full),
            pl.BlockSpec((H2, H), full), pl.BlockSpec((1, H2), full),
        ],
        out_specs=[
            pl.BlockSpec((BN, H), row), pl.BlockSpec((BN, H), row),
            pl.BlockSpec((BN, H2), row),
        ],
        out_shape=[
            jax.ShapeDtypeStruct((n, H), jnp.float32),
            jax.ShapeDtypeStruct((n, H), jnp.float32),
            jax.ShapeDtypeStruct((n, H2), jnp.float32),
        ],
    )(nf, in1, rl1[0], rl1[1].reshape(1, H), rl2[0], rl2[1].reshape(1, H),
      wi, bi.reshape(1, H), wab, bab.reshape(1, H2))


def _edge_body(pa, pb, wt, bt, ft_o):
    h = _lrelu(pa[...] + pb[...])
    t = _dg(h, wt[...]) + bt[...]
    k = jax.nn.sigmoid(t[:, H2:H2 + 1])
    ft_o[...] = jnp.transpose(t[:, :H2] * k)


def _edge(pre_a, pre_b, wt, bt):
    e = pre_a.shape[0]
    grid = (e // BE,)
    row = lambda i: (i, 0)
    col = lambda i: (0, i)
    full = lambda i: (0, 0)
    return pl.pallas_call(
        _edge_body,
        grid=grid,
        in_specs=[
            pl.BlockSpec((BE, H2), row), pl.BlockSpec((BE, H2), row),
            pl.BlockSpec((640, H2), full), pl.BlockSpec((1, 640), full),
        ],
        out_specs=pl.BlockSpec((H2, BE), col),
        out_shape=jax.ShapeDtypeStruct((H2, e), jnp.float32),
    )(pre_a, pre_b, wt, bt)


def _post_body(x_dst, f1st, mxt, x_in1,
               wr1, wr2, wr3, br, wg, bg, wpc1, wpc2, bpc, out_o):
    nfno2t = jnp.where(mxt[...] == NEG_INF, 0.0, mxt[...])
    new_x = (_dg(x_dst[...], wr1[...]) + _dgt(f1st[...], wr2[...]) +
             _dgt(nfno2t, wr3[...]) + br[...])
    new_x = _dg(new_x, wg[...]) + bg[...]
    out_o[...] = (x_dst[...] + _dg(new_x, wpc1[...]) +
                  _dg(x_in1[...], wpc2[...]) + bpc[...])


def _post(x_dst, f1st, mxt, x_in1, red, gw, gb, pcw, pcb):
    n = x_dst.shape[0]
    grid = (pl.cdiv(n, BN),)
    row = lambda i: (i, 0)
    col = lambda i: (0, i)
    full = lambda i: (0, 0)
    wred, bred = red
    return pl.pallas_call(
        _post_body,
        grid=grid,
        in_specs=[
            pl.BlockSpec((BN, H), row), pl.BlockSpec((H, BN), col),
            pl.BlockSpec((H, BN), col), pl.BlockSpec((BN, H), row),
            pl.BlockSpec((H, H), full), pl.BlockSpec((H, H), full),
            pl.BlockSpec((H, H), full), pl.BlockSpec((1, H), full),
            pl.BlockSpec((H, H), full), pl.BlockSpec((1, H), full),
            pl.BlockSpec((H, H), full), pl.BlockSpec((H, H), full),
            pl.BlockSpec((1, H), full),
        ],
        out_specs=pl.BlockSpec((BN, H), row),
        out_shape=jax.ShapeDtypeStruct((n, H), jnp.float32),
    )(x_dst, f1st, mxt, x_in1,
      wred[:, :H], wred[:, H:2 * H], wred[:, 2 * H:], bred.reshape(1, H),
      gw, gb.reshape(1, H), pcw[:, :H], pcw[:, H:], pcb.reshape(1, H))


def _mid_body(xn, xc, nl1w, nl1b, nl2w, nl2b, cl1w, cl1b, cl2w, cl2b,
              w1s, w1d, b1, xn_o, a_o, xc_o, b_o):
    t = _dg(xn[...], nl1w[...]) + nl1b[...]
    xnn = _dg(t, nl2w[...]) + nl2b[...] + xn[...]
    xn_o[...] = xnn
    a_o[...] = _dg(xnn, w1s[...])
    t2 = _dg(xc[...], cl1w[...]) + cl1b[...]
    xcc = _dg(t2, cl2w[...]) + cl2b[...] + xc[...]
    xc_o[...] = xcc
    b_o[...] = _dg(xcc, w1d[...]) + b1[...]


def _mid(x_gn_mid, x_gc, res_gn2, res_gc2, w1s, w1d, b1):
    n = x_gn_mid.shape[0]
    grid = (pl.cdiv(n, BN),)
    row = lambda i: (i, 0)
    full = lambda i: (0, 0)
    return pl.pallas_call(
        _mid_body,
        grid=grid,
        in_specs=[
            pl.BlockSpec((BN, H), row), pl.BlockSpec((BN, H), row),
            pl.BlockSpec((H, H), full), pl.BlockSpec((1, H), full),
            pl.BlockSpec((H, H), full), pl.BlockSpec((1, H), full),
            pl.BlockSpec((H, H), full), pl.BlockSpec((1, H), full),
            pl.BlockSpec((H, H), full), pl.BlockSpec((1, H), full),
            pl.BlockSpec((H2, H), full), pl.BlockSpec((H2, H), full),
            pl.BlockSpec((1, H2), full),
        ],
        out_specs=[
            pl.BlockSpec((BN, H), row), pl.BlockSpec((BN, H2), row),
            pl.BlockSpec((BN, H), row), pl.BlockSpec((BN, H2), row),
        ],
        out_shape=[
            jax.ShapeDtypeStruct((n, H), jnp.float32),
            jax.ShapeDtypeStruct((n, H2), jnp.float32),
            jax.ShapeDtypeStruct((n, H), jnp.float32),
            jax.ShapeDtypeStruct((n, H2), jnp.float32),
        ],
    )(x_gn_mid, x_gc,
      res_gn2['l1'][0], res_gn2['l1'][1].reshape(1, H),
      res_gn2['l2'][0], res_gn2['l2'][1].reshape(1, H),
      res_gc2['l1'][0], res_gc2['l1'][1].reshape(1, H),
      res_gc2['l2'][0], res_gc2['l2'][1].reshape(1, H),
      w1s, w1d, b1.reshape(1, H2))


def _sc_gather(a_tab, b_tab, src, dst):
    e = src.shape[0]
    ep = e // SC_WORKERS
    chunks = ep // GC
    mesh = plsc.VectorSubcoreMesh(core_axis_name="c", subcore_axis_name="s")

    @functools.partial(
        pl.kernel, mesh=mesh,
        out_type=[
            jax.ShapeDtypeStruct((e, H2), jnp.float32),
            jax.ShapeDtypeStruct((e, H2), jnp.float32),
        ],
        compiler_params=pltpu.CompilerParams(needs_layout_passes=False),
        scratch_types=[
            pltpu.VMEM((GC,), jnp.int32),
            pltpu.VMEM((GC,), jnp.int32),
            pltpu.VMEM((GC, H2), jnp.float32),
            pltpu.VMEM((GC, H2), jnp.float32),
            pltpu.SemaphoreType.DMA,
            pltpu.SemaphoreType.DMA,
            pltpu.SemaphoreType.DMA,
        ],
    )
    def k(a_h, b_h, src_h, dst_h, pa_h, pb_h, ia, ib, ra, rb, semi, sem,
          semw):
        c = lax.axis_index("c")
        s = lax.axis_index("s")
        wid = s * SC_CORES + c

        def chunk(i, carry):
            base = pl.multiple_of(wid * ep + i * GC, 8)
            ld1 = pltpu.async_copy(src_h.at[pl.ds(base, GC)], ia, semi)
            ld2 = pltpu.async_copy(dst_h.at[pl.ds(base, GC)], ib, semi)
            ld1.wait()
            ld2.wait()
            cp1 = pltpu.async_copy(a_h.at[ia], ra, sem)
            cp2 = pltpu.async_copy(b_h.at[ib], rb, sem)
            cp1.wait()
            cp2.wait()
            wr1 = pltpu.async_copy(ra, pa_h.at[pl.ds(base, GC)], semw)
            wr2 = pltpu.async_copy(rb, pb_h.at[pl.ds(base, GC)], semw)
            wr1.wait()
            wr2.wait()
            return carry

        lax.fori_loop(0, chunks, chunk, 0)

    return k(a_tab, b_tab, src, dst)


def _sc_scatter_sum(ftt, dst, n):
    e = dst.shape[0]
    chunks = e // MCH
    groups = MCH // 16
    mesh = plsc.VectorSubcoreMesh(core_axis_name="c", subcore_axis_name="s")

    @functools.partial(
        pl.kernel, mesh=mesh,
        out_type=jax.ShapeDtypeStruct((H * n,), jnp.float32),
        compiler_params=pltpu.CompilerParams(needs_layout_passes=False),
        scratch_types=[
            pltpu.VMEM((8 * n,), jnp.float32),
            pltpu.VMEM((n,), jnp.int32),
            pltpu.VMEM((MCH,), jnp.int32),
            pltpu.VMEM((8, MCH), jnp.float32),
        ],
    )
    def k(ftt_h, dst_h, out_h, acc, tag, midx, mval):
        c = lax.axis_index("c")
        s = lax.axis_index("s")
        tid = c * SC_SUBCORES + s
        iota = lax.iota(jnp.int32, 16)

        def minit(i, carry):
            acc[pl.ds(i * 16, 16)] = jnp.zeros((16,), jnp.float32)
            return carry
        lax.fori_loop(0, (8 * n) // 16, minit, 0)

        def ch_loop(ch, carry):
            base = ch * MCH
            pltpu.sync_copy(dst_h.at[pl.ds(base, MCH)], midx)
            pltpu.sync_copy(
                ftt_h.at[pl.ds(H + tid * 8, 8), pl.ds(base, MCH)], mval)

            def grp(q, carry3):
                for u in range(4):
                    g = q * 4 + u
                    dstv = midx[pl.ds(g * 16, 16)]
                    for f in range(8):
                        plsc.addupdate_scatter(
                            acc, [dstv + f * n], mval[f, pl.ds(g * 16, 16)])
                return carry3
            lax.fori_loop(0, groups // 4, grp, 0)
            return carry
        lax.fori_loop(0, chunks, ch_loop, 0)

        for f in range(8):
            pltpu.sync_copy(acc.at[pl.ds(f * n, n)],
                            out_h.at[pl.ds((tid * 8 + f) * n, n)])

    return k(ftt, dst)


def _sc_scatter_max(ftt, dst, n):
    e = dst.shape[0]
    chunks = e // MCH
    groups = MCH // 16
    mesh = plsc.VectorSubcoreMesh(core_axis_name="c", subcore_axis_name="s")

    @functools.partial(
        pl.kernel, mesh=mesh,
        out_type=jax.ShapeDtypeStruct((H * n,), jnp.float32),
        compiler_params=pltpu.CompilerParams(needs_layout_passes=False),
        scratch_types=[
            pltpu.VMEM((8 * n,), jnp.float32),
            pltpu.VMEM((n,), jnp.int32),
            pltpu.VMEM((MCH,), jnp.int32),
            pltpu.VMEM((8, MCH), jnp.float32),
        ],
    )
    def k(ftt_h, dst_h, out_h, acc, tag, midx, mval):
        c = lax.axis_index("c")
        s = lax.axis_index("s")
        tid = c * SC_SUBCORES + s
        iota = lax.iota(jnp.int32, 16)

        def minit(i, carry):
            acc[pl.ds(i * 16, 16)] = jnp.full((16,), NEG_INF, jnp.float32)
            return carry
        lax.fori_loop(0, (8 * n) // 16, minit, 0)

        def ch_loop(ch, carry):
            base = ch * MCH
            pltpu.sync_copy(dst_h.at[pl.ds(base, MCH)], midx)
            pltpu.sync_copy(
                ftt_h.at[pl.ds(tid * 8, 8), pl.ds(base, MCH)], mval)

            def grp(g, carry3):
                dstv = midx[pl.ds(g * 16, 16)]
                plsc.store_scatter(tag, [dstv], iota)
                dup_free = jnp.all(plsc.load_gather(tag, [dstv]) == iota)

                @pl.when(dup_free)
                def _():
                    for f in range(8):
                        val = mval[f, pl.ds(g * 16, 16)]
                        aidx = dstv + f * n
                        cur = plsc.load_gather(acc, [aidx])
                        plsc.store_scatter(acc, [aidx],
                                           jnp.maximum(cur, val))

                @pl.when(jnp.logical_not(dup_free))
                def _():
                    for f in range(8):
                        val = mval[f, pl.ds(g * 16, 16)]
                        aidx = dstv + f * n
                        cur = plsc.load_gather(acc, [aidx])
                        m0 = val > cur

                        def wcond(m):
                            return jnp.any(m)

                        def wbody(m):
                            plsc.store_scatter(acc, [aidx], val, mask=m)
                            cur2 = plsc.load_gather(acc, [aidx])
                            return m & (cur2 < val)

                        lax.while_loop(wcond, wbody, m0)
                return carry3
            lax.fori_loop(0, groups, grp, 0)
            return carry
        lax.fori_loop(0, chunks, ch_loop, 0)

        for f in range(8):
            pltpu.sync_copy(acc.at[pl.ds(f * n, n)],
                            out_h.at[pl.ds((tid * 8 + f) * n, n)])

    return k(ftt, dst)


def _direction(a_tab, b_tab, x_dst, x_in1, edge, msg, red, gw, gb, pcw, pcb,
               n_dst):
    w2, b2 = msg['l2']
    wt = jnp.concatenate(
        [w2[1 + H:], w2[1:1 + H], w2[0:1],
         jnp.zeros((127, H2), jnp.float32)], axis=0)
    bt = jnp.concatenate(
        [b2[1 + H:], b2[1:1 + H], b2[0:1],
         jnp.zeros((127,), jnp.float32)]).reshape(1, 640)
    src, dst = edge[0], edge[1]
    pre_a, pre_b = _sc_gather(a_tab, b_tab, src, dst)
    ftt = _edge(pre_a, pre_b, wt, bt)
    f1st = _sc_scatter_sum(ftt, dst, n_dst).reshape(H, n_dst)
    mxt = _sc_scatter_max(ftt, dst, n_dst).reshape(H, n_dst)
    return _post(x_dst, f1st, mxt, x_in1, red, gw, gb, pcw, pcb)


def kernel(nf_gc, nf_gn, nf_gc_in1, nf_gn_in1, edge_c2n, edge_n2c, params):
    p = params
    w1c, b1c = p['msg_c2n']['l1']
    w1n, b1n = p['msg_n2c']['l1']

    x_gc, x_gc_in1, a_c2n = _prep(
        nf_gc, nf_gc_in1, p['res_gc_1']['l1'], p['res_gc_1']['l2'],
        p['gc_in1'][0], p['gc_in1'][1], w1c[:, :H],
        jnp.zeros((H2,), jnp.float32))
    x_gn, x_gn_in1, b_c2n = _prep(
        nf_gn, nf_gn_in1, p['res_gn_1']['l1'], p['res_gn_1']['l2'],
        p['gn_in1'][0], p['gn_in1'][1], w1c[:, H:], b1c)

    x_gn_mid = _direction(a_c2n, b_c2n, x_gn, x_gn_in1, edge_c2n,
                          p['msg_c2n'], p['red_c2n'], p['Gcn'][0],
                          p['Gcn'][1], p['postCatGcn'][0], p['postCatGcn'][1],
                          nf_gn.shape[0])

    x_gn2, a_n2c, x_gc2, b_n2c = _mid(
        x_gn_mid, x_gc, p['res_gn_2'], p['res_gc_2'],
        w1n[:, :H], w1n[:, H:], b1n)

    x_gc_out = _direction(a_n2c, b_n2c, x_gc2, x_gc_in1, edge_n2c,
                          p['msg_n2c'], p['red_n2c'], p['Gnc'][0],
                          p['Gnc'][1], p['postCatGnc'][0], p['postCatGnc'][1],
                          nf_gc.shape[0])

    return (x_gc_out, x_gn2)


# speedup vs baseline: 1.2126x; 1.0330x over previous
import functools

import jax
import jax.numpy as jnp
from jax import lax
from jax.experimental import pallas as pl
from jax.experimental.pallas import tpu as pltpu
from jax.experimental.pallas import tpu_sc as plsc

H = 256
H2 = 512
NEG_INF = float("-inf")

BN = 512
BE = 640

SC_CORES = 2
SC_SUBCORES = 16
SC_WORKERS = SC_CORES * SC_SUBCORES
GC = 80
SUMC = 40
MSUPER = 6400
MCH = 1280
NROW = 624
ZROW = 48


def _dg(x, w):
    return lax.dot_general(x, w, (((1,), (1,)), ((), ())),
                           preferred_element_type=jnp.float32)


def _dgt(xt, w):
    return lax.dot_general(xt, w, (((0,), (1,)), ((), ())),
                           preferred_element_type=jnp.float32)


def _lrelu(x):
    return jnp.where(x >= 0.0, x, 0.2 * x)


def _prep_body(nf, in1, wl1, bl1, wl2, bl2, wi, bi, wab, bab,
               x_o, xin1_o, ab_o):
    t = _dg(nf[...], wl1[...]) + bl1[...]
    x = _dg(t, wl2[...]) + bl2[...] + nf[...]
    x_o[...] = x
    xin1_o[...] = _dg(in1[...], wi[...]) + bi[...]
    ab_o[...] = _dg(x, wab[...]) + bab[...]


def _prep(nf, in1, rl1, rl2, wi, bi, wab, bab):
    n = nf.shape[0]
    grid = (pl.cdiv(n, BN),)
    row = lambda i: (i, 0)
    full = lambda i: (0, 0)
    return pl.pallas_call(
        _prep_body,
        grid=grid,
        in_specs=[
            pl.BlockSpec((BN, H), row), pl.BlockSpec((BN, H), row),
            pl.BlockSpec((H, H), full), pl.BlockSpec((1, H), full),
            pl.BlockSpec((H, H), full), pl.BlockSpec((1, H), full),
            pl.BlockSpec((H, H), full), pl.BlockSpec((1, H), full),
            pl.BlockSpec((H2, H), full), pl.BlockSpec((1, H2), full),
        ],
        out_specs=[
            pl.BlockSpec((BN, H), row), pl.BlockSpec((BN, H), row),
            pl.BlockSpec((BN, H2), row),
        ],
        out_shape=[
            jax.ShapeDtypeStruct((n, H), jnp.float32),
            jax.ShapeDtypeStruct((n, H), jnp.float32),
            jax.ShapeDtypeStruct((n, H2), jnp.float32),
        ],
    )(nf, in1, rl1[0], rl1[1].reshape(1, H), rl2[0], rl2[1].reshape(1, H),
      wi, bi.reshape(1, H), wab, bab.reshape(1, H2))


def _edge_body(pa, pb, wt, bt, ft_o):
    h = _lrelu(pa[...] + pb[...])
    t = _dg(h, wt[...]) + bt[...]
    k = jax.nn.sigmoid(t[:, H2:H2 + 1])
    ft_o[...] = jnp.transpose(t[:, :H2] * k)


def _edge(pre_a, pre_b, wt, bt):
    e = pre_a.shape[0]
    grid = (e // BE,)
    row = lambda i: (i, 0)
    col = lambda i: (0, i)
    full = lambda i: (0, 0)
    return pl.pallas_call(
        _edge_body,
        grid=grid,
        in_specs=[
            pl.BlockSpec((BE, H2), row), pl.BlockSpec((BE, H2), row),
            pl.BlockSpec((640, H2), full), pl.BlockSpec((1, 640), full),
        ],
        out_specs=pl.BlockSpec((H2, BE), col),
        out_shape=jax.ShapeDtypeStruct((H2, e), jnp.float32),
    )(pre_a, pre_b, wt, bt)


def _post_body(x_dst, f1st, mxt, x_in1,
               wr1, wr2, wr3, br, wg, bg, wpc1, wpc2, bpc, out_o):
    nfno2t = jnp.where(mxt[...] == NEG_INF, 0.0, mxt[...])
    new_x = (_dg(x_dst[...], wr1[...]) + _dgt(f1st[...], wr2[...]) +
             _dgt(nfno2t, wr3[...]) + br[...])
    new_x = _dg(new_x, wg[...]) + bg[...]
    out_o[...] = (x_dst[...] + _dg(new_x, wpc1[...]) +
                  _dg(x_in1[...], wpc2[...]) + bpc[...])


def _post(x_dst, f1st, mxt, x_in1, red, gw, gb, pcw, pcb):
    n = x_dst.shape[0]
    grid = (pl.cdiv(n, BN),)
    row = lambda i: (i, 0)
    col = lambda i: (0, i)
    full = lambda i: (0, 0)
    wred, bred = red
    return pl.pallas_call(
        _post_body,
        grid=grid,
        in_specs=[
            pl.BlockSpec((BN, H), row), pl.BlockSpec((H, BN), col),
            pl.BlockSpec((H, BN), col), pl.BlockSpec((BN, H), row),
            pl.BlockSpec((H, H), full), pl.BlockSpec((H, H), full),
            pl.BlockSpec((H, H), full), pl.BlockSpec((1, H), full),
            pl.BlockSpec((H, H), full), pl.BlockSpec((1, H), full),
            pl.BlockSpec((H, H), full), pl.BlockSpec((H, H), full),
            pl.BlockSpec((1, H), full),
        ],
        out_specs=pl.BlockSpec((BN, H), row),
        out_shape=jax.ShapeDtypeStruct((n, H), jnp.float32),
    )(x_dst, f1st, mxt, x_in1,
      wred[:, :H], wred[:, H:2 * H], wred[:, 2 * H:], bred.reshape(1, H),
      gw, gb.reshape(1, H), pcw[:, :H], pcw[:, H:], pcb.reshape(1, H))


def _mid_body(xn, xc, nl1w, nl1b, nl2w, nl2b, cl1w, cl1b, cl2w, cl2b,
              w1s, w1d, b1, xn_o, a_o, xc_o, b_o):
    t = _dg(xn[...], nl1w[...]) + nl1b[...]
    xnn = _dg(t, nl2w[...]) + nl2b[...] + xn[...]
    xn_o[...] = xnn
    a_o[...] = _dg(xnn, w1s[...])
    t2 = _dg(xc[...], cl1w[...]) + cl1b[...]
    xcc = _dg(t2, cl2w[...]) + cl2b[...] + xc[...]
    xc_o[...] = xcc
    b_o[...] = _dg(xcc, w1d[...]) + b1[...]


def _mid(x_gn_mid, x_gc, res_gn2, res_gc2, w1s, w1d, b1):
    n = x_gn_mid.shape[0]
    grid = (pl.cdiv(n, BN),)
    row = lambda i: (i, 0)
    full = lambda i: (0, 0)
    return pl.pallas_call(
        _mid_body,
        grid=grid,
        in_specs=[
            pl.BlockSpec((BN, H), row), pl.BlockSpec((BN, H), row),
            pl.BlockSpec((H, H), full), pl.BlockSpec((1, H), full),
            pl.BlockSpec((H, H), full), pl.BlockSpec((1, H), full),
            pl.BlockSpec((H, H), full), pl.BlockSpec((1, H), full),
            pl.BlockSpec((H, H), full), pl.BlockSpec((1, H), full),
            pl.BlockSpec((H2, H), full), pl.BlockSpec((H2, H), full),
            pl.BlockSpec((1, H2), full),
        ],
        out_specs=[
            pl.BlockSpec((BN, H), row), pl.BlockSpec((BN, H2), row),
            pl.BlockSpec((BN, H), row), pl.BlockSpec((BN, H2), row),
        ],
        out_shape=[
            jax.ShapeDtypeStruct((n, H), jnp.float32),
            jax.ShapeDtypeStruct((n, H2), jnp.float32),
            jax.ShapeDtypeStruct((n, H), jnp.float32),
            jax.ShapeDtypeStruct((n, H2), jnp.float32),
        ],
    )(x_gn_mid, x_gc,
      res_gn2['l1'][0], res_gn2['l1'][1].reshape(1, H),
      res_gn2['l2'][0], res_gn2['l2'][1].reshape(1, H),
      res_gc2['l1'][0], res_gc2['l1'][1].reshape(1, H),
      res_gc2['l2'][0], res_gc2['l2'][1].reshape(1, H),
      w1s, w1d, b1.reshape(1, H2))


def _sc_gather(a_tab, b_tab, src, dst):
    e = src.shape[0]
    nch = e // GC
    iters = (nch + SC_WORKERS - 1) // SC_WORKERS
    mesh = plsc.VectorSubcoreMesh(core_axis_name="c", subcore_axis_name="s")

    @functools.partial(
        pl.kernel, mesh=mesh,
        out_type=[
            jax.ShapeDtypeStruct((e, H2), jnp.float32),
            jax.ShapeDtypeStruct((e, H2), jnp.float32),
        ],
        compiler_params=pltpu.CompilerParams(needs_layout_passes=False),
        scratch_types=[
            pltpu.VMEM((GC,), jnp.int32),
            pltpu.VMEM((GC,), jnp.int32),
            pltpu.VMEM((GC, H2), jnp.float32),
            pltpu.VMEM((GC, H2), jnp.float32),
            pltpu.SemaphoreType.DMA,
            pltpu.SemaphoreType.DMA,
            pltpu.SemaphoreType.DMA,
        ],
    )
    def k(a_h, b_h, src_h, dst_h, pa_h, pb_h, ia, ib, ra, rb, semi, sem,
          semw):
        c = lax.axis_index("c")
        s = lax.axis_index("s")
        wid = s * SC_CORES + c

        def chunk(i, carry):
            j = i * SC_WORKERS + wid
            j = jnp.where(j < nch, j, 0)
            base = pl.multiple_of(j * GC, 8)
            ld1 = pltpu.async_copy(src_h.at[pl.ds(base, GC)], ia, semi)
            ld2 = pltpu.async_copy(dst_h.at[pl.ds(base, GC)], ib, semi)
            ld1.wait()
            ld2.wait()
            cp1 = pltpu.async_copy(a_h.at[ia], ra, sem)
            cp2 = pltpu.async_copy(b_h.at[ib], rb, sem)
            cp1.wait()
            cp2.wait()
            wr1 = pltpu.async_copy(ra, pa_h.at[pl.ds(base, GC)], semw)
            wr2 = pltpu.async_copy(rb, pb_h.at[pl.ds(base, GC)], semw)
            wr1.wait()
            wr2.wait()
            return carry

        lax.fori_loop(0, iters, chunk, 0)

    return k(a_tab, b_tab, src, dst)


def _sc_scatter_sum(ftt, dst, n):
    e = dst.shape[0]
    chunks = e // MCH
    groups = MCH // 16
    mesh = plsc.VectorSubcoreMesh(core_axis_name="c", subcore_axis_name="s")

    @functools.partial(
        pl.kernel, mesh=mesh,
        out_type=jax.ShapeDtypeStruct((H * n,), jnp.float32),
        compiler_params=pltpu.CompilerParams(needs_layout_passes=False),
        scratch_types=[
            pltpu.VMEM((8 * n,), jnp.float32),
            pltpu.VMEM((n,), jnp.int32),
            pltpu.VMEM((MCH,), jnp.int32),
            pltpu.VMEM((8, MCH), jnp.float32),
        ],
    )
    def k(ftt_h, dst_h, out_h, acc, tag, midx, mval):
        c = lax.axis_index("c")
        s = lax.axis_index("s")
        tid = c * SC_SUBCORES + s
        iota = lax.iota(jnp.int32, 16)

        def minit(i, carry):
            acc[pl.ds(i * 16, 16)] = jnp.zeros((16,), jnp.float32)
            return carry
        lax.fori_loop(0, (8 * n) // 16, minit, 0)

        def ch_loop(ch, carry):
            base = ch * MCH
            pltpu.sync_copy(dst_h.at[pl.ds(base, MCH)], midx)
            pltpu.sync_copy(
                ftt_h.at[pl.ds(H + tid * 8, 8), pl.ds(base, MCH)], mval)

            def grp(q, carry3):
                for u in range(4):
                    g = q * 4 + u
                    dstv = midx[pl.ds(g * 16, 16)]
                    for f in range(8):
                        plsc.addupdate_scatter(
                            acc, [dstv + f * n], mval[f, pl.ds(g * 16, 16)])
                return carry3
            lax.fori_loop(0, groups // 4, grp, 0)
            return carry
        lax.fori_loop(0, chunks, ch_loop, 0)

        for f in range(8):
            pltpu.sync_copy(acc.at[pl.ds(f * n, n)],
                            out_h.at[pl.ds((tid * 8 + f) * n, n)])

    return k(ftt, dst)


def _sc_scatter_max(ftt, dst, n):
    e = dst.shape[0]
    chunks = e // MCH
    groups = MCH // 16
    mesh = plsc.VectorSubcoreMesh(core_axis_name="c", subcore_axis_name="s")

    @functools.partial(
        pl.kernel, mesh=mesh,
        out_type=jax.ShapeDtypeStruct((H * n,), jnp.float32),
        compiler_params=pltpu.CompilerParams(needs_layout_passes=False),
        scratch_types=[
            pltpu.VMEM((8 * n,), jnp.float32),
            pltpu.VMEM((n,), jnp.int32),
            pltpu.VMEM((MCH,), jnp.int32),
            pltpu.VMEM((8, MCH), jnp.float32),
        ],
    )
    def k(ftt_h, dst_h, out_h, acc, tag, midx, mval):
        c = lax.axis_index("c")
        s = lax.axis_index("s")
        tid = c * SC_SUBCORES + s
        iota = lax.iota(jnp.int32, 16)

        def minit(i, carry):
            acc[pl.ds(i * 16, 16)] = jnp.full((16,), NEG_INF, jnp.float32)
            return carry
        lax.fori_loop(0, (8 * n) // 16, minit, 0)

        def ch_loop(ch, carry):
            base = ch * MCH
            pltpu.sync_copy(dst_h.at[pl.ds(base, MCH)], midx)
            pltpu.sync_copy(
                ftt_h.at[pl.ds(tid * 8, 8), pl.ds(base, MCH)], mval)

            def grp(g, carry3):
                dstv = midx[pl.ds(g * 16, 16)]
                plsc.store_scatter(tag, [dstv], iota)
                dup_free = jnp.all(plsc.load_gather(tag, [dstv]) == iota)

                @pl.when(dup_free)
                def _():
                    for f in range(8):
                        val = mval[f, pl.ds(g * 16, 16)]
                        aidx = dstv + f * n
                        cur = plsc.load_gather(acc, [aidx])
                        plsc.store_scatter(acc, [aidx],
                                           jnp.maximum(cur, val))

                @pl.when(jnp.logical_not(dup_free))
                def _():
                    for f in range(8):
                        val = mval[f, pl.ds(g * 16, 16)]
                        aidx = dstv + f * n
                        cur = plsc.load_gather(acc, [aidx])
                        m0 = val > cur

                        def wcond(m):
                            return jnp.any(m)

                        def wbody(m):
                            plsc.store_scatter(acc, [aidx], val, mask=m)
                            cur2 = plsc.load_gather(acc, [aidx])
                            return m & (cur2 < val)

                        lax.while_loop(wcond, wbody, m0)
                return carry3
            lax.fori_loop(0, groups, grp, 0)
            return carry
        lax.fori_loop(0, chunks, ch_loop, 0)

        for f in range(8):
            pltpu.sync_copy(acc.at[pl.ds(f * n, n)],
                            out_h.at[pl.ds((tid * 8 + f) * n, n)])

    return k(ftt, dst)


def _direction(a_tab, b_tab, x_dst, x_in1, edge, msg, red, gw, gb, pcw, pcb,
               n_dst):
    w2, b2 = msg['l2']
    wt = jnp.concatenate(
        [w2[1 + H:], w2[1:1 + H], w2[0:1],
         jnp.zeros((127, H2), jnp.float32)], axis=0)
    bt = jnp.concatenate(
        [b2[1 + H:], b2[1:1 + H], b2[0:1],
         jnp.zeros((127,), jnp.float32)]).reshape(1, 640)
    src, dst = edge[0], edge[1]
    pre_a, pre_b = _sc_gather(a_tab, b_tab, src, dst)
    ftt = _edge(pre_a, pre_b, wt, bt)
    f1st = _sc_scatter_sum(ftt, dst, n_dst).reshape(H, n_dst)
    mxt = _sc_scatter_max(ftt, dst, n_dst).reshape(H, n_dst)
    return _post(x_dst, f1st, mxt, x_in1, red, gw, gb, pcw, pcb)


def kernel(nf_gc, nf_gn, nf_gc_in1, nf_gn_in1, edge_c2n, edge_n2c, params):
    p = params
    w1c, b1c = p['msg_c2n']['l1']
    w1n, b1n = p['msg_n2c']['l1']

    x_gc, x_gc_in1, a_c2n = _prep(
        nf_gc, nf_gc_in1, p['res_gc_1']['l1'], p['res_gc_1']['l2'],
        p['gc_in1'][0], p['gc_in1'][1], w1c[:, :H],
        jnp.zeros((H2,), jnp.float32))
    x_gn, x_gn_in1, b_c2n = _prep(
        nf_gn, nf_gn_in1, p['res_gn_1']['l1'], p['res_gn_1']['l2'],
        p['gn_in1'][0], p['gn_in1'][1], w1c[:, H:], b1c)

    x_gn_mid = _direction(a_c2n, b_c2n, x_gn, x_gn_in1, edge_c2n,
                          p['msg_c2n'], p['red_c2n'], p['Gcn'][0],
                          p['Gcn'][1], p['postCatGcn'][0], p['postCatGcn'][1],
                          nf_gn.shape[0])

    x_gn2, a_n2c, x_gc2, b_n2c = _mid(
        x_gn_mid, x_gc, p['res_gn_2'], p['res_gc_2'],
        w1n[:, :H], w1n[:, H:], b1n)

    x_gc_out = _direction(a_n2c, b_n2c, x_gc2, x_gc_in1, edge_n2c,
                          p['msg_n2c'], p['red_n2c'], p['Gnc'][0],
                          p['Gnc'][1], p['postCatGnc'][0], p['postCatGnc'][1],
                          nf_gc.shape[0])

    return (x_gc_out, x_gn2)


# speedup vs baseline: 1.4312x; 1.1803x over previous
import functools

import jax
import jax.numpy as jnp
from jax import lax
from jax.experimental import pallas as pl
from jax.experimental.pallas import tpu as pltpu
from jax.experimental.pallas import tpu_sc as plsc

H = 256
H2 = 512
NEG_INF = float("-inf")

BN = 512
BE = 640

SC_CORES = 2
SC_SUBCORES = 16
SC_WORKERS = SC_CORES * SC_SUBCORES
GC = 80
SUMC = 40
MSUPER = 6400
MCH = 1280
NROW = 624
ZROW = 48


def _dg(x, w):
    return lax.dot_general(x, w, (((1,), (1,)), ((), ())),
                           preferred_element_type=jnp.float32)


def _dgt(xt, w):
    return lax.dot_general(xt, w, (((0,), (1,)), ((), ())),
                           preferred_element_type=jnp.float32)


def _lrelu(x):
    return jnp.where(x >= 0.0, x, 0.2 * x)


def _prep_body(nf, in1, wl1, bl1, wl2, bl2, wi, bi, wab, bab,
               x_o, xin1_o, ab_o):
    t = _dg(nf[...], wl1[...]) + bl1[...]
    x = _dg(t, wl2[...]) + bl2[...] + nf[...]
    x_o[...] = x
    xin1_o[...] = _dg(in1[...], wi[...]) + bi[...]
    ab_o[...] = _dg(x, wab[...]) + bab[...]


def _prep(nf, in1, rl1, rl2, wi, bi, wab, bab):
    n = nf.shape[0]
    grid = (pl.cdiv(n, BN),)
    row = lambda i: (i, 0)
    full = lambda i: (0, 0)
    return pl.pallas_call(
        _prep_body,
        grid=grid,
        in_specs=[
            pl.BlockSpec((BN, H), row), pl.BlockSpec((BN, H), row),
            pl.BlockSpec((H, H), full), pl.BlockSpec((1, H), full),
            pl.BlockSpec((H, H), full), pl.BlockSpec((1, H), full),
            pl.BlockSpec((H, H), full), pl.BlockSpec((1, H), full),
            pl.BlockSpec((H2, H), full), pl.BlockSpec((1, H2), full),
        ],
        out_specs=[
            pl.BlockSpec((BN, H), row), pl.BlockSpec((BN, H), row),
            pl.BlockSpec((BN, H2), row),
        ],
        out_shape=[
            jax.ShapeDtypeStruct((n, H), jnp.float32),
            jax.ShapeDtypeStruct((n, H), jnp.float32),
            jax.ShapeDtypeStruct((n, H2), jnp.float32),
        ],
    )(nf, in1, rl1[0], rl1[1].reshape(1, H), rl2[0], rl2[1].reshape(1, H),
      wi, bi.reshape(1, H), wab, bab.reshape(1, H2))


def _edge_body(pa, pb, wt, bt, ft_o):
    h = _lrelu(pa[...] + pb[...])
    t = _dg(h, wt[...]) + bt[...]
    k = jax.nn.sigmoid(t[:, H2:H2 + 1])
    ft_o[...] = jnp.transpose(t[:, :H2] * k)


def _edge(pre_a, pre_b, wt, bt):
    e = pre_a.shape[0]
    grid = (e // BE,)
    row = lambda i: (i, 0)
    col = lambda i: (0, i)
    full = lambda i: (0, 0)
    return pl.pallas_call(
        _edge_body,
        grid=grid,
        in_specs=[
            pl.BlockSpec((BE, H2), row), pl.BlockSpec((BE, H2), row),
            pl.BlockSpec((640, H2), full), pl.BlockSpec((1, 640), full),
        ],
        out_specs=pl.BlockSpec((H2, BE), col),
        out_shape=jax.ShapeDtypeStruct((H2, e), jnp.float32),
    )(pre_a, pre_b, wt, bt)


def _post_body(x_dst, f1st, mxt, x_in1,
               wr1, wr2, wr3, br, wg, bg, wpc1, wpc2, bpc, out_o):
    nfno2t = jnp.where(mxt[...] == NEG_INF, 0.0, mxt[...])
    new_x = (_dg(x_dst[...], wr1[...]) + _dgt(f1st[...], wr2[...]) +
             _dgt(nfno2t, wr3[...]) + br[...])
    new_x = _dg(new_x, wg[...]) + bg[...]
    out_o[...] = (x_dst[...] + _dg(new_x, wpc1[...]) +
                  _dg(x_in1[...], wpc2[...]) + bpc[...])


def _post(x_dst, f1st, mxt, x_in1, red, gw, gb, pcw, pcb):
    n = x_dst.shape[0]
    grid = (pl.cdiv(n, BN),)
    row = lambda i: (i, 0)
    col = lambda i: (0, i)
    full = lambda i: (0, 0)
    wred, bred = red
    return pl.pallas_call(
        _post_body,
        grid=grid,
        in_specs=[
            pl.BlockSpec((BN, H), row), pl.BlockSpec((H, BN), col),
            pl.BlockSpec((H, BN), col), pl.BlockSpec((BN, H), row),
            pl.BlockSpec((H, H), full), pl.BlockSpec((H, H), full),
            pl.BlockSpec((H, H), full), pl.BlockSpec((1, H), full),
            pl.BlockSpec((H, H), full), pl.BlockSpec((1, H), full),
            pl.BlockSpec((H, H), full), pl.BlockSpec((H, H), full),
            pl.BlockSpec((1, H), full),
        ],
        out_specs=pl.BlockSpec((BN, H), row),
        out_shape=jax.ShapeDtypeStruct((n, H), jnp.float32),
    )(x_dst, f1st, mxt, x_in1,
      wred[:, :H], wred[:, H:2 * H], wred[:, 2 * H:], bred.reshape(1, H),
      gw, gb.reshape(1, H), pcw[:, :H], pcw[:, H:], pcb.reshape(1, H))


def _mid_body(xn, xc, nl1w, nl1b, nl2w, nl2b, cl1w, cl1b, cl2w, cl2b,
              w1s, w1d, b1, xn_o, a_o, xc_o, b_o):
    t = _dg(xn[...], nl1w[...]) + nl1b[...]
    xnn = _dg(t, nl2w[...]) + nl2b[...] + xn[...]
    xn_o[...] = xnn
    a_o[...] = _dg(xnn, w1s[...])
    t2 = _dg(xc[...], cl1w[...]) + cl1b[...]
    xcc = _dg(t2, cl2w[...]) + cl2b[...] + xc[...]
    xc_o[...] = xcc
    b_o[...] = _dg(xcc, w1d[...]) + b1[...]


def _mid(x_gn_mid, x_gc, res_gn2, res_gc2, w1s, w1d, b1):
    n = x_gn_mid.shape[0]
    grid = (pl.cdiv(n, BN),)
    row = lambda i: (i, 0)
    full = lambda i: (0, 0)
    return pl.pallas_call(
        _mid_body,
        grid=grid,
        in_specs=[
            pl.BlockSpec((BN, H), row), pl.BlockSpec((BN, H), row),
            pl.BlockSpec((H, H), full), pl.BlockSpec((1, H), full),
            pl.BlockSpec((H, H), full), pl.BlockSpec((1, H), full),
            pl.BlockSpec((H, H), full), pl.BlockSpec((1, H), full),
            pl.BlockSpec((H, H), full), pl.BlockSpec((1, H), full),
            pl.BlockSpec((H2, H), full), pl.BlockSpec((H2, H), full),
            pl.BlockSpec((1, H2), full),
        ],
        out_specs=[
            pl.BlockSpec((BN, H), row), pl.BlockSpec((BN, H2), row),
            pl.BlockSpec((BN, H), row), pl.BlockSpec((BN, H2), row),
        ],
        out_shape=[
            jax.ShapeDtypeStruct((n, H), jnp.float32),
            jax.ShapeDtypeStruct((n, H2), jnp.float32),
            jax.ShapeDtypeStruct((n, H), jnp.float32),
            jax.ShapeDtypeStruct((n, H2), jnp.float32),
        ],
    )(x_gn_mid, x_gc,
      res_gn2['l1'][0], res_gn2['l1'][1].reshape(1, H),
      res_gn2['l2'][0], res_gn2['l2'][1].reshape(1, H),
      res_gc2['l1'][0], res_gc2['l1'][1].reshape(1, H),
      res_gc2['l2'][0], res_gc2['l2'][1].reshape(1, H),
      w1s, w1d, b1.reshape(1, H2))


def _sc_gather(a_tab, b_tab, src, dst):
    e = src.shape[0]
    nch = e // GC
    iters = (nch + SC_WORKERS - 1) // SC_WORKERS
    mesh = plsc.VectorSubcoreMesh(core_axis_name="c", subcore_axis_name="s")

    @functools.partial(
        pl.kernel, mesh=mesh,
        out_type=[
            jax.ShapeDtypeStruct((e, H2), jnp.float32),
            jax.ShapeDtypeStruct((e, H2), jnp.float32),
        ],
        compiler_params=pltpu.CompilerParams(needs_layout_passes=False),
        scratch_types=[
            pltpu.VMEM((GC,), jnp.int32),
            pltpu.VMEM((GC,), jnp.int32),
            pltpu.VMEM((GC, H2), jnp.float32),
            pltpu.VMEM((GC, H2), jnp.float32),
            pltpu.SemaphoreType.DMA,
            pltpu.SemaphoreType.DMA,
            pltpu.SemaphoreType.DMA,
        ],
    )
    def k(a_h, b_h, src_h, dst_h, pa_h, pb_h, ia, ib, ra, rb, semi, sem,
          semw):
        c = lax.axis_index("c")
        s = lax.axis_index("s")
        wid = s * SC_CORES + c

        def chunk(i, carry):
            j = i * SC_WORKERS + wid
            j = jnp.where(j < nch, j, 0)
            base = pl.multiple_of(j * GC, 8)
            ld1 = pltpu.async_copy(src_h.at[pl.ds(base, GC)], ia, semi)
            ld2 = pltpu.async_copy(dst_h.at[pl.ds(base, GC)], ib, semi)
            ld1.wait()
            ld2.wait()
            cp1 = pltpu.async_copy(a_h.at[ia], ra, sem)
            cp2 = pltpu.async_copy(b_h.at[ib], rb, sem)
            cp1.wait()
            cp2.wait()
            wr1 = pltpu.async_copy(ra, pa_h.at[pl.ds(base, GC)], semw)
            wr2 = pltpu.async_copy(rb, pb_h.at[pl.ds(base, GC)], semw)
            wr1.wait()
            wr2.wait()
            return carry

        lax.fori_loop(0, iters, chunk, 0)

    return k(a_tab, b_tab, src, dst)


def _sc_scatter_sum(ftt, dst, n):
    e = dst.shape[0]
    chunks = e // MCH
    groups = MCH // 16
    mesh = plsc.VectorSubcoreMesh(core_axis_name="c", subcore_axis_name="s")

    @functools.partial(
        pl.kernel, mesh=mesh,
        out_type=jax.ShapeDtypeStruct((H * n,), jnp.float32),
        compiler_params=pltpu.CompilerParams(needs_layout_passes=False),
        scratch_types=[
            pltpu.VMEM((8 * n,), jnp.float32),
            pltpu.VMEM((MCH,), jnp.int32),
            pltpu.VMEM((8, MCH), jnp.float32),
            pltpu.VMEM((MCH,), jnp.int32),
            pltpu.VMEM((8, MCH), jnp.float32),
            pltpu.SemaphoreType.DMA,
            pltpu.SemaphoreType.DMA,
        ],
    )
    def k(ftt_h, dst_h, out_h, acc, midx0, mval0, midx1, mval1, sem0, sem1):
        c = lax.axis_index("c")
        s = lax.axis_index("s")
        tid = c * SC_SUBCORES + s

        def minit(i, carry):
            acc[pl.ds(i * 16, 16)] = jnp.zeros((16,), jnp.float32)
            return carry
        lax.fori_loop(0, (8 * n) // 16, minit, 0)

        def start(ch, midx, mval, sem):
            base = ch * MCH
            c1 = pltpu.async_copy(dst_h.at[pl.ds(base, MCH)], midx, sem)
            c2 = pltpu.async_copy(
                ftt_h.at[pl.ds(H + tid * 8, 8), pl.ds(base, MCH)], mval, sem)
            return c1, c2

        def process(midx, mval):
            def grp(q, carry3):
                for u in range(4):
                    g = q * 4 + u
                    dstv = midx[pl.ds(g * 16, 16)]
                    for f in range(8):
                        plsc.addupdate_scatter(
                            acc, [dstv + f * n], mval[f, pl.ds(g * 16, 16)])
                return carry3
            lax.fori_loop(0, groups // 4, grp, 0)

        a1, a2 = start(0, midx0, mval0, sem0)
        a1.wait()
        a2.wait()

        def ch2_loop(q, carry):
            b1, b2 = start(2 * q + 1, midx1, mval1, sem1)
            process(midx0, mval0)
            b1.wait()
            b2.wait()
            a1, a2 = start(2 * q + 2, midx0, mval0, sem0)
            process(midx1, mval1)
            a1.wait()
            a2.wait()
            return carry
        lax.fori_loop(0, (chunks - 1) // 2, ch2_loop, 0)
        process(midx0, mval0)

        for f in range(8):
            pltpu.sync_copy(acc.at[pl.ds(f * n, n)],
                            out_h.at[pl.ds((tid * 8 + f) * n, n)])

    return k(ftt, dst)


def _sc_scatter_max(ftt, dst, n):
    e = dst.shape[0]
    chunks = e // MCH
    groups = MCH // 16
    mesh = plsc.VectorSubcoreMesh(core_axis_name="c", subcore_axis_name="s")

    @functools.partial(
        pl.kernel, mesh=mesh,
        out_type=jax.ShapeDtypeStruct((H * n,), jnp.float32),
        compiler_params=pltpu.CompilerParams(needs_layout_passes=False),
        scratch_types=[
            pltpu.VMEM((8 * n,), jnp.float32),
            pltpu.VMEM((n,), jnp.int32),
            pltpu.VMEM((MCH,), jnp.int32),
            pltpu.VMEM((8, MCH), jnp.float32),
            pltpu.VMEM((MCH,), jnp.int32),
            pltpu.VMEM((8, MCH), jnp.float32),
            pltpu.SemaphoreType.DMA,
            pltpu.SemaphoreType.DMA,
        ],
    )
    def k(ftt_h, dst_h, out_h, acc, tag, midx0, mval0, midx1, mval1,
          sem0, sem1):
        c = lax.axis_index("c")
        s = lax.axis_index("s")
        tid = c * SC_SUBCORES + s
        iota = lax.iota(jnp.int32, 16)

        def minit(i, carry):
            acc[pl.ds(i * 16, 16)] = jnp.full((16,), NEG_INF, jnp.float32)
            return carry
        lax.fori_loop(0, (8 * n) // 16, minit, 0)

        def start(ch, midx, mval, sem):
            base = ch * MCH
            c1 = pltpu.async_copy(dst_h.at[pl.ds(base, MCH)], midx, sem)
            c2 = pltpu.async_copy(
                ftt_h.at[pl.ds(tid * 8, 8), pl.ds(base, MCH)], mval, sem)
            return c1, c2

        def process(midx, mval):
            def grp(g, carry3):
                dstv = midx[pl.ds(g * 16, 16)]
                plsc.store_scatter(tag, [dstv], iota)
                dup_free = jnp.all(plsc.load_gather(tag, [dstv]) == iota)

                @pl.when(dup_free)
                def _():
                    for f in range(8):
                        val = mval[f, pl.ds(g * 16, 16)]
                        aidx = dstv + f * n
                        cur = plsc.load_gather(acc, [aidx])
                        plsc.store_scatter(acc, [aidx],
                                           jnp.maximum(cur, val))

                @pl.when(jnp.logical_not(dup_free))
                def _():
                    for f in range(8):
                        val = mval[f, pl.ds(g * 16, 16)]
                        aidx = dstv + f * n
                        cur = plsc.load_gather(acc, [aidx])
                        m0 = val > cur

                        def wcond(m):
                            return jnp.any(m)

                        def wbody(m):
                            plsc.store_scatter(acc, [aidx], val, mask=m)
                            cur2 = plsc.load_gather(acc, [aidx])
                            return m & (cur2 < val)

                        lax.while_loop(wcond, wbody, m0)
                return carry3
            lax.fori_loop(0, groups, grp, 0)

        a1, a2 = start(0, midx0, mval0, sem0)
        a1.wait()
        a2.wait()

        def ch2_loop(q, carry):
            b1, b2 = start(2 * q + 1, midx1, mval1, sem1)
            process(midx0, mval0)
            b1.wait()
            b2.wait()
            a1, a2 = start(2 * q + 2, midx0, mval0, sem0)
            process(midx1, mval1)
            a1.wait()
            a2.wait()
            return carry
        lax.fori_loop(0, (chunks - 1) // 2, ch2_loop, 0)
        process(midx0, mval0)

        for f in range(8):
            pltpu.sync_copy(acc.at[pl.ds(f * n, n)],
                            out_h.at[pl.ds((tid * 8 + f) * n, n)])

    return k(ftt, dst)


def _direction(a_tab, b_tab, x_dst, x_in1, edge, msg, red, gw, gb, pcw, pcb,
               n_dst):
    w2, b2 = msg['l2']
    wt = jnp.concatenate(
        [w2[1 + H:], w2[1:1 + H], w2[0:1],
         jnp.zeros((127, H2), jnp.float32)], axis=0)
    bt = jnp.concatenate(
        [b2[1 + H:], b2[1:1 + H], b2[0:1],
         jnp.zeros((127,), jnp.float32)]).reshape(1, 640)
    src, dst = edge[0], edge[1]
    pre_a, pre_b = _sc_gather(a_tab, b_tab, src, dst)
    ftt = _edge(pre_a, pre_b, wt, bt)
    f1st = _sc_scatter_sum(ftt, dst, n_dst).reshape(H, n_dst)
    mxt = _sc_scatter_max(ftt, dst, n_dst).reshape(H, n_dst)
    return _post(x_dst, f1st, mxt, x_in1, red, gw, gb, pcw, pcb)


def kernel(nf_gc, nf_gn, nf_gc_in1, nf_gn_in1, edge_c2n, edge_n2c, params):
    p = params
    w1c, b1c = p['msg_c2n']['l1']
    w1n, b1n = p['msg_n2c']['l1']

    x_gc, x_gc_in1, a_c2n = _prep(
        nf_gc, nf_gc_in1, p['res_gc_1']['l1'], p['res_gc_1']['l2'],
        p['gc_in1'][0], p['gc_in1'][1], w1c[:, :H],
        jnp.zeros((H2,), jnp.float32))
    x_gn, x_gn_in1, b_c2n = _prep(
        nf_gn, nf_gn_in1, p['res_gn_1']['l1'], p['res_gn_1']['l2'],
        p['gn_in1'][0], p['gn_in1'][1], w1c[:, H:], b1c)

    x_gn_mid = _direction(a_c2n, b_c2n, x_gn, x_gn_in1, edge_c2n,
                          p['msg_c2n'], p['red_c2n'], p['Gcn'][0],
                          p['Gcn'][1], p['postCatGcn'][0], p['postCatGcn'][1],
                          nf_gn.shape[0])

    x_gn2, a_n2c, x_gc2, b_n2c = _mid(
        x_gn_mid, x_gc, p['res_gn_2'], p['res_gc_2'],
        w1n[:, :H], w1n[:, H:], b1n)

    x_gc_out = _direction(a_n2c, b_n2c, x_gc2, x_gc_in1, edge_n2c,
                          p['msg_n2c'], p['red_n2c'], p['Gnc'][0],
                          p['Gnc'][1], p['postCatGnc'][0], p['postCatGnc'][1],
                          nf_gc.shape[0])

    return (x_gc_out, x_gn2)


# speedup vs baseline: 1.4710x; 1.0278x over previous
import functools

import jax
import jax.numpy as jnp
from jax import lax
from jax.experimental import pallas as pl
from jax.experimental.pallas import tpu as pltpu
from jax.experimental.pallas import tpu_sc as plsc

H = 256
H2 = 512
NEG_INF = float("-inf")

BN = 512
BE = 640

SC_CORES = 2
SC_SUBCORES = 16
SC_WORKERS = SC_CORES * SC_SUBCORES
GC = 40
SUMC = 40
MSUPER = 6400
MCH = 1280
NROW = 624
ZROW = 48


def _dg(x, w):
    return lax.dot_general(x, w, (((1,), (1,)), ((), ())),
                           preferred_element_type=jnp.float32)


def _dgt(xt, w):
    return lax.dot_general(xt, w, (((0,), (1,)), ((), ())),
                           preferred_element_type=jnp.float32)


def _lrelu(x):
    return jnp.where(x >= 0.0, x, 0.2 * x)


def _prep_body(nf, in1, wl1, bl1, wl2, bl2, wi, bi, wab, bab,
               x_o, xin1_o, ab_o):
    t = _dg(nf[...], wl1[...]) + bl1[...]
    x = _dg(t, wl2[...]) + bl2[...] + nf[...]
    x_o[...] = x
    xin1_o[...] = _dg(in1[...], wi[...]) + bi[...]
    ab_o[...] = _dg(x, wab[...]) + bab[...]


def _prep(nf, in1, rl1, rl2, wi, bi, wab, bab):
    n = nf.shape[0]
    grid = (pl.cdiv(n, BN),)
    row = lambda i: (i, 0)
    full = lambda i: (0, 0)
    return pl.pallas_call(
        _prep_body,
        grid=grid,
        in_specs=[
            pl.BlockSpec((BN, H), row), pl.BlockSpec((BN, H), row),
            pl.BlockSpec((H, H), full), pl.BlockSpec((1, H), full),
            pl.BlockSpec((H, H), full), pl.BlockSpec((1, H), full),
            pl.BlockSpec((H, H), full), pl.BlockSpec((1, H), full),
            pl.BlockSpec((H2, H), full), pl.BlockSpec((1, H2), full),
        ],
        out_specs=[
            pl.BlockSpec((BN, H), row), pl.BlockSpec((BN, H), row),
            pl.BlockSpec((BN, H2), row),
        ],
        out_shape=[
            jax.ShapeDtypeStruct((n, H), jnp.float32),
            jax.ShapeDtypeStruct((n, H), jnp.float32),
            jax.ShapeDtypeStruct((n, H2), jnp.float32),
        ],
    )(nf, in1, rl1[0], rl1[1].reshape(1, H), rl2[0], rl2[1].reshape(1, H),
      wi, bi.reshape(1, H), wab, bab.reshape(1, H2))


def _edge_body(pa, pb, wt, bt, ft_o):
    h = _lrelu(pa[...] + pb[...])
    t = _dg(h, wt[...]) + bt[...]
    k = jax.nn.sigmoid(t[:, H2:H2 + 1])
    ft_o[...] = jnp.transpose(t[:, :H2] * k)


def _edge(pre_a, pre_b, wt, bt):
    e = pre_a.shape[0]
    grid = (e // BE,)
    row = lambda i: (i, 0)
    col = lambda i: (0, i)
    full = lambda i: (0, 0)
    return pl.pallas_call(
        _edge_body,
        grid=grid,
        in_specs=[
            pl.BlockSpec((BE, H2), row), pl.BlockSpec((BE, H2), row),
            pl.BlockSpec((640, H2), full), pl.BlockSpec((1, 640), full),
        ],
        out_specs=pl.BlockSpec((H2, BE), col),
        out_shape=jax.ShapeDtypeStruct((H2, e), jnp.float32),
    )(pre_a, pre_b, wt, bt)


def _post_body(x_dst, f1st, mxt, x_in1,
               wr1, wr2, wr3, br, wg, bg, wpc1, wpc2, bpc, out_o):
    nfno2t = jnp.where(mxt[...] == NEG_INF, 0.0, mxt[...])
    new_x = (_dg(x_dst[...], wr1[...]) + _dgt(f1st[...], wr2[...]) +
             _dgt(nfno2t, wr3[...]) + br[...])
    new_x = _dg(new_x, wg[...]) + bg[...]
    out_o[...] = (x_dst[...] + _dg(new_x, wpc1[...]) +
                  _dg(x_in1[...], wpc2[...]) + bpc[...])


def _post(x_dst, f1st, mxt, x_in1, red, gw, gb, pcw, pcb):
    n = x_dst.shape[0]
    grid = (pl.cdiv(n, BN),)
    row = lambda i: (i, 0)
    col = lambda i: (0, i)
    full = lambda i: (0, 0)
    wred, bred = red
    return pl.pallas_call(
        _post_body,
        grid=grid,
        in_specs=[
            pl.BlockSpec((BN, H), row), pl.BlockSpec((H, BN), col),
            pl.BlockSpec((H, BN), col), pl.BlockSpec((BN, H), row),
            pl.BlockSpec((H, H), full), pl.BlockSpec((H, H), full),
            pl.BlockSpec((H, H), full), pl.BlockSpec((1, H), full),
            pl.BlockSpec((H, H), full), pl.BlockSpec((1, H), full),
            pl.BlockSpec((H, H), full), pl.BlockSpec((H, H), full),
            pl.BlockSpec((1, H), full),
        ],
        out_specs=pl.BlockSpec((BN, H), row),
        out_shape=jax.ShapeDtypeStruct((n, H), jnp.float32),
    )(x_dst, f1st, mxt, x_in1,
      wred[:, :H], wred[:, H:2 * H], wred[:, 2 * H:], bred.reshape(1, H),
      gw, gb.reshape(1, H), pcw[:, :H], pcw[:, H:], pcb.reshape(1, H))


def _mid_body(xn, xc, nl1w, nl1b, nl2w, nl2b, cl1w, cl1b, cl2w, cl2b,
              w1s, w1d, b1, xn_o, a_o, xc_o, b_o):
    t = _dg(xn[...], nl1w[...]) + nl1b[...]
    xnn = _dg(t, nl2w[...]) + nl2b[...] + xn[...]
    xn_o[...] = xnn
    a_o[...] = _dg(xnn, w1s[...])
    t2 = _dg(xc[...], cl1w[...]) + cl1b[...]
    xcc = _dg(t2, cl2w[...]) + cl2b[...] + xc[...]
    xc_o[...] = xcc
    b_o[...] = _dg(xcc, w1d[...]) + b1[...]


def _mid(x_gn_mid, x_gc, res_gn2, res_gc2, w1s, w1d, b1):
    n = x_gn_mid.shape[0]
    grid = (pl.cdiv(n, BN),)
    row = lambda i: (i, 0)
    full = lambda i: (0, 0)
    return pl.pallas_call(
        _mid_body,
        grid=grid,
        in_specs=[
            pl.BlockSpec((BN, H), row), pl.BlockSpec((BN, H), row),
            pl.BlockSpec((H, H), full), pl.BlockSpec((1, H), full),
            pl.BlockSpec((H, H), full), pl.BlockSpec((1, H), full),
            pl.BlockSpec((H, H), full), pl.BlockSpec((1, H), full),
            pl.BlockSpec((H, H), full), pl.BlockSpec((1, H), full),
            pl.BlockSpec((H2, H), full), pl.BlockSpec((H2, H), full),
            pl.BlockSpec((1, H2), full),
        ],
        out_specs=[
            pl.BlockSpec((BN, H), row), pl.BlockSpec((BN, H2), row),
            pl.BlockSpec((BN, H), row), pl.BlockSpec((BN, H2), row),
        ],
        out_shape=[
            jax.ShapeDtypeStruct((n, H), jnp.float32),
            jax.ShapeDtypeStruct((n, H2), jnp.float32),
            jax.ShapeDtypeStruct((n, H), jnp.float32),
            jax.ShapeDtypeStruct((n, H2), jnp.float32),
        ],
    )(x_gn_mid, x_gc,
      res_gn2['l1'][0], res_gn2['l1'][1].reshape(1, H),
      res_gn2['l2'][0], res_gn2['l2'][1].reshape(1, H),
      res_gc2['l1'][0], res_gc2['l1'][1].reshape(1, H),
      res_gc2['l2'][0], res_gc2['l2'][1].reshape(1, H),
      w1s, w1d, b1.reshape(1, H2))


def _sc_gather(a_tab, b_tab, src, dst):
    e = src.shape[0]
    ep = e // SC_WORKERS
    chunks = ep // GC
    mesh = plsc.VectorSubcoreMesh(core_axis_name="c", subcore_axis_name="s")

    @functools.partial(
        pl.kernel, mesh=mesh,
        out_type=[
            jax.ShapeDtypeStruct((e, H2), jnp.float32),
            jax.ShapeDtypeStruct((e, H2), jnp.float32),
        ],
        compiler_params=pltpu.CompilerParams(needs_layout_passes=False),
        scratch_types=[
            pltpu.VMEM((ep,), jnp.int32),
            pltpu.VMEM((ep,), jnp.int32),
            pltpu.VMEM((GC, H2), jnp.float32),
            pltpu.VMEM((GC, H2), jnp.float32),
            pltpu.VMEM((GC, H2), jnp.float32),
            pltpu.VMEM((GC, H2), jnp.float32),
            pltpu.SemaphoreType.DMA,
            pltpu.SemaphoreType.DMA,
            pltpu.SemaphoreType.DMA,
            pltpu.SemaphoreType.DMA,
        ],
    )
    def k(a_h, b_h, src_h, dst_h, pa_h, pb_h, ia, ib, ra0, rb0, ra1, rb1,
          semi, semg0, semg1, semw):
        c = lax.axis_index("c")
        s = lax.axis_index("s")
        wid = s * SC_CORES + c
        tbase = pl.multiple_of(wid * ep, 8)

        l1 = pltpu.async_copy(src_h.at[pl.ds(tbase, ep)], ia, semi)
        l2 = pltpu.async_copy(dst_h.at[pl.ds(tbase, ep)], ib, semi)
        l1.wait()
        l2.wait()

        def gstart(ch, ra, rb, sem):
            off = ch * GC
            c1 = pltpu.async_copy(a_h.at[ia.at[pl.ds(off, GC)]], ra, sem)
            c2 = pltpu.async_copy(b_h.at[ib.at[pl.ds(off, GC)]], rb, sem)
            return c1, c2

        def wstart(ch, ra, rb):
            base = pl.multiple_of(tbase + ch * GC, 8)
            c1 = pltpu.async_copy(ra, pa_h.at[pl.ds(base, GC)], semw)
            c2 = pltpu.async_copy(rb, pb_h.at[pl.ds(base, GC)], semw)
            return c1, c2

        g1, g2 = gstart(0, ra0, rb0, semg0)
        g1.wait()
        g2.wait()

        def loop(q, carry):
            b1, b2 = gstart(2 * q + 1, ra1, rb1, semg1)
            w1, w2 = wstart(2 * q, ra0, rb0)
            b1.wait()
            b2.wait()
            w1.wait()
            w2.wait()
            a1, a2 = gstart(2 * q + 2, ra0, rb0, semg0)
            w3, w4 = wstart(2 * q + 1, ra1, rb1)
            a1.wait()
            a2.wait()
            w3.wait()
            w4.wait()
            return carry
        lax.fori_loop(0, (chunks - 1) // 2, loop, 0)

        w1, w2 = wstart(chunks - 1, ra0, rb0)
        w1.wait()
        w2.wait()

    return k(a_tab, b_tab, src, dst)


def _sc_scatter_sum(ftt, dst, n):
    e = dst.shape[0]
    chunks = e // MCH
    groups = MCH // 16
    mesh = plsc.VectorSubcoreMesh(core_axis_name="c", subcore_axis_name="s")

    @functools.partial(
        pl.kernel, mesh=mesh,
        out_type=jax.ShapeDtypeStruct((H * n,), jnp.float32),
        compiler_params=pltpu.CompilerParams(needs_layout_passes=False),
        scratch_types=[
            pltpu.VMEM((8 * n,), jnp.float32),
            pltpu.VMEM((MCH,), jnp.int32),
            pltpu.VMEM((8, MCH), jnp.float32),
            pltpu.VMEM((MCH,), jnp.int32),
            pltpu.VMEM((8, MCH), jnp.float32),
            pltpu.SemaphoreType.DMA,
            pltpu.SemaphoreType.DMA,
        ],
    )
    def k(ftt_h, dst_h, out_h, acc, midx0, mval0, midx1, mval1, sem0, sem1):
        c = lax.axis_index("c")
        s = lax.axis_index("s")
        tid = c * SC_SUBCORES + s

        def minit(i, carry):
            acc[pl.ds(i * 16, 16)] = jnp.zeros((16,), jnp.float32)
            return carry
        lax.fori_loop(0, (8 * n) // 16, minit, 0)

        def start(ch, midx, mval, sem):
            base = ch * MCH
            c1 = pltpu.async_copy(dst_h.at[pl.ds(base, MCH)], midx, sem)
            c2 = pltpu.async_copy(
                ftt_h.at[pl.ds(H + tid * 8, 8), pl.ds(base, MCH)], mval, sem)
            return c1, c2

        def process(midx, mval):
            def grp(q, carry3):
                for u in range(4):
                    g = q * 4 + u
                    dstv = midx[pl.ds(g * 16, 16)]
                    for f in range(8):
                        plsc.addupdate_scatter(
                            acc, [dstv + f * n], mval[f, pl.ds(g * 16, 16)])
                return carry3
            lax.fori_loop(0, groups // 4, grp, 0)

        a1, a2 = start(0, midx0, mval0, sem0)
        a1.wait()
        a2.wait()

        def ch2_loop(q, carry):
            b1, b2 = start(2 * q + 1, midx1, mval1, sem1)
            process(midx0, mval0)
            b1.wait()
            b2.wait()
            a1, a2 = start(2 * q + 2, midx0, mval0, sem0)
            process(midx1, mval1)
            a1.wait()
            a2.wait()
            return carry
        lax.fori_loop(0, (chunks - 1) // 2, ch2_loop, 0)
        process(midx0, mval0)

        for f in range(8):
            pltpu.sync_copy(acc.at[pl.ds(f * n, n)],
                            out_h.at[pl.ds((tid * 8 + f) * n, n)])

    return k(ftt, dst)


def _sc_scatter_max(ftt, dst, n):
    e = dst.shape[0]
    chunks = e // MCH
    groups = MCH // 16
    mesh = plsc.VectorSubcoreMesh(core_axis_name="c", subcore_axis_name="s")

    @functools.partial(
        pl.kernel, mesh=mesh,
        out_type=jax.ShapeDtypeStruct((H * n,), jnp.float32),
        compiler_params=pltpu.CompilerParams(needs_layout_passes=False),
        scratch_types=[
            pltpu.VMEM((8 * n,), jnp.float32),
            pltpu.VMEM((n,), jnp.int32),
            pltpu.VMEM((MCH,), jnp.int32),
            pltpu.VMEM((8, MCH), jnp.float32),
            pltpu.VMEM((MCH,), jnp.int32),
            pltpu.VMEM((8, MCH), jnp.float32),
            pltpu.SemaphoreType.DMA,
            pltpu.SemaphoreType.DMA,
        ],
    )
    def k(ftt_h, dst_h, out_h, acc, tag, midx0, mval0, midx1, mval1,
          sem0, sem1):
        c = lax.axis_index("c")
        s = lax.axis_index("s")
        tid = c * SC_SUBCORES + s
        iota = lax.iota(jnp.int32, 16)

        def minit(i, carry):
            acc[pl.ds(i * 16, 16)] = jnp.full((16,), NEG_INF, jnp.float32)
            return carry
        lax.fori_loop(0, (8 * n) // 16, minit, 0)

        def start(ch, midx, mval, sem):
            base = ch * MCH
            c1 = pltpu.async_copy(dst_h.at[pl.ds(base, MCH)], midx, sem)
            c2 = pltpu.async_copy(
                ftt_h.at[pl.ds(tid * 8, 8), pl.ds(base, MCH)], mval, sem)
            return c1, c2

        def process(midx, mval):
            def grp(g, carry3):
                dstv = midx[pl.ds(g * 16, 16)]
                plsc.store_scatter(tag, [dstv], iota)
                dup_free = jnp.all(plsc.load_gather(tag, [dstv]) == iota)

                @pl.when(dup_free)
                def _():
                    for f in range(8):
                        val = mval[f, pl.ds(g * 16, 16)]
                        aidx = dstv + f * n
                        cur = plsc.load_gather(acc, [aidx])
                        plsc.store_scatter(acc, [aidx],
                                           jnp.maximum(cur, val))

                @pl.when(jnp.logical_not(dup_free))
                def _():
                    for f in range(8):
                        val = mval[f, pl.ds(g * 16, 16)]
                        aidx = dstv + f * n
                        cur = plsc.load_gather(acc, [aidx])
                        m0 = val > cur

                        def wcond(m):
                            return jnp.any(m)

                        def wbody(m):
                            plsc.store_scatter(acc, [aidx], val, mask=m)
                            cur2 = plsc.load_gather(acc, [aidx])
                            return m & (cur2 < val)

                        lax.while_loop(wcond, wbody, m0)
                return carry3
            lax.fori_loop(0, groups, grp, 0)

        a1, a2 = start(0, midx0, mval0, sem0)
        a1.wait()
        a2.wait()

        def ch2_loop(q, carry):
            b1, b2 = start(2 * q + 1, midx1, mval1, sem1)
            process(midx0, mval0)
            b1.wait()
            b2.wait()
            a1, a2 = start(2 * q + 2, midx0, mval0, sem0)
            process(midx1, mval1)
            a1.wait()
            a2.wait()
            return carry
        lax.fori_loop(0, (chunks - 1) // 2, ch2_loop, 0)
        process(midx0, mval0)

        for f in range(8):
            pltpu.sync_copy(acc.at[pl.ds(f * n, n)],
                            out_h.at[pl.ds((tid * 8 + f) * n, n)])

    return k(ftt, dst)


def _direction(a_tab, b_tab, x_dst, x_in1, edge, msg, red, gw, gb, pcw, pcb,
               n_dst):
    w2, b2 = msg['l2']
    wt = jnp.concatenate(
        [w2[1 + H:], w2[1:1 + H], w2[0:1],
         jnp.zeros((127, H2), jnp.float32)], axis=0)
    bt = jnp.concatenate(
        [b2[1 + H:], b2[1:1 + H], b2[0:1],
         jnp.zeros((127,), jnp.float32)]).reshape(1, 640)
    src, dst = edge[0], edge[1]
    pre_a, pre_b = _sc_gather(a_tab, b_tab, src, dst)
    ftt = _edge(pre_a, pre_b, wt, bt)
    f1st = _sc_scatter_sum(ftt, dst, n_dst).reshape(H, n_dst)
    mxt = _sc_scatter_max(ftt, dst, n_dst).reshape(H, n_dst)
    return _post(x_dst, f1st, mxt, x_in1, red, gw, gb, pcw, pcb)


def kernel(nf_gc, nf_gn, nf_gc_in1, nf_gn_in1, edge_c2n, edge_n2c, params):
    p = params
    w1c, b1c = p['msg_c2n']['l1']
    w1n, b1n = p['msg_n2c']['l1']

    x_gc, x_gc_in1, a_c2n = _prep(
        nf_gc, nf_gc_in1, p['res_gc_1']['l1'], p['res_gc_1']['l2'],
        p['gc_in1'][0], p['gc_in1'][1], w1c[:, :H],
        jnp.zeros((H2,), jnp.float32))
    x_gn, x_gn_in1, b_c2n = _prep(
        nf_gn, nf_gn_in1, p['res_gn_1']['l1'], p['res_gn_1']['l2'],
        p['gn_in1'][0], p['gn_in1'][1], w1c[:, H:], b1c)

    x_gn_mid = _direction(a_c2n, b_c2n, x_gn, x_gn_in1, edge_c2n,
                          p['msg_c2n'], p['red_c2n'], p['Gcn'][0],
                          p['Gcn'][1], p['postCatGcn'][0], p['postCatGcn'][1],
                          nf_gn.shape[0])

    x_gn2, a_n2c, x_gc2, b_n2c = _mid(
        x_gn_mid, x_gc, p['res_gn_2'], p['res_gc_2'],
        w1n[:, :H], w1n[:, H:], b1n)

    x_gc_out = _direction(a_n2c, b_n2c, x_gc2, x_gc_in1, edge_n2c,
                          p['msg_n2c'], p['red_n2c'], p['Gnc'][0],
                          p['Gnc'][1], p['postCatGnc'][0], p['postCatGnc'][1],
                          nf_gc.shape[0])

    return (x_gc_out, x_gn2)
